# baseline (speedup 1.0000x reference)
import functools

import jax
import jax.numpy as jnp
from jax import lax
from jax.experimental import pallas as pl
from jax.experimental.pallas import tpu as pltpu
from jax.experimental.pallas import tpu_sc as plsc

_N = 10000
_E = 320000
_D = 128
_LN_EPS = 1e-5

_BE = 2000
_BN = 2000
_NW = 32
_GCH = 80
_SCH = 80


def _ln_silu(h, g, b):
    mu = jnp.mean(h, axis=-1, keepdims=True)
    c = h - mu
    var = jnp.mean(c * c, axis=-1, keepdims=True)
    hn = c * lax.rsqrt(var + _LN_EPS) * g + b
    return hn * jax.nn.sigmoid(hn)


def _const_spec(shape):
    return pl.BlockSpec(shape, lambda i: tuple(0 for _ in shape))


def _linear_pl(x, w, b, block):
    n, k = x.shape
    m = w.shape[1]
    def body(x_ref, w_ref, b_ref, o_ref):
        o_ref[...] = (
            jnp.dot(x_ref[...].astype(jnp.bfloat16), w_ref[...], preferred_element_type=jnp.float32)
            + b_ref[...])
    return pl.pallas_call(
        body,
        grid=(n // block,),
        in_specs=[pl.BlockSpec((block, k), lambda i: (i, 0)),
                  _const_spec((k, m)),
                  _const_spec((1, m))],
        out_specs=pl.BlockSpec((block, m), lambda i: (i, 0)),
        out_shape=jax.ShapeDtypeStruct((n, m), jnp.float32),
    )(x, w, b.reshape(1, m))


def _message_pl(g, e, wd, ws, we, b1, g1, t1, w2m, b2m, w2g, b2g):
    nb = _E // _BE

    def body(hd_ref, hs_ref, e_ref, wd_, ws_, we_, b1_, g1_, t1_,
             w2m_, b2m_, w2g_, b2g_, o_ref):
        h1 = (jnp.dot(hd_ref[...].astype(jnp.bfloat16), wd_[...], preferred_element_type=jnp.float32)
              + jnp.dot(hs_ref[...].astype(jnp.bfloat16), ws_[...], preferred_element_type=jnp.float32)
              + jnp.dot(e_ref[...].astype(jnp.bfloat16), we_[...], preferred_element_type=jnp.float32)
              + b1_[...])
        hm = _ln_silu(h1[:, :_D], g1_[:, :_D], t1_[:, :_D])
        hg = _ln_silu(h1[:, _D:], g1_[:, _D:], t1_[:, _D:])
        msg = jnp.dot(hm.astype(jnp.bfloat16), w2m_[...], preferred_element_type=jnp.float32) + b2m_[...]
        gl = jnp.dot(hg.astype(jnp.bfloat16), w2g_[...], preferred_element_type=jnp.float32) + b2g_[...]
        o_ref[...] = jax.nn.sigmoid(gl) * msg

    return pl.pallas_call(
        body,
        grid=(nb,),
        in_specs=[pl.BlockSpec((_BE, _D), lambda i: (i + nb, 0)),
                  pl.BlockSpec((_BE, _D), lambda i: (i, 0)),
                  pl.BlockSpec((_BE, _D), lambda i: (i, 0)),
                  _const_spec((_D, 2 * _D)), _const_spec((_D, 2 * _D)),
                  _const_spec((_D, 2 * _D)), _const_spec((1, 2 * _D)),
                  _const_spec((1, 2 * _D)), _const_spec((1, 2 * _D)),
                  _const_spec((_D, _D)), _const_spec((1, _D)),
                  _const_spec((_D, _D)), _const_spec((1, _D))],
        out_specs=pl.BlockSpec((_BE, _D), lambda i: (i, 0)),
        out_shape=jax.ShapeDtypeStruct((_E, _D), jnp.float32),
    )(g, g, e, wd, ws, we, b1, g1, t1, w2m, b2m, w2g, b2g)


def _node_pl(h, parts, wh, wa, b1, g1, t1, w2, b2):
    nb = _N // _BN

    def body(h_ref, p0_ref, p1_ref, wh_, wa_, b1_, g1_, t1_, w2_, b2_, o_ref):
        aggr = p0_ref[...] + p1_ref[...]
        h1 = (jnp.dot(h_ref[...].astype(jnp.bfloat16), wh_[...], preferred_element_type=jnp.float32)
              + jnp.dot(aggr.astype(jnp.bfloat16), wa_[...], preferred_element_type=jnp.float32)
              + b1_[...])
        h1 = _ln_silu(h1, g1_[...], t1_[...])
        o_ref[...] = (jnp.dot(h1.astype(jnp.bfloat16), w2_[...], preferred_element_type=jnp.float32)
                      + b2_[...] + h_ref[...])

    return pl.pallas_call(
        body,
        grid=(nb,),
        in_specs=[pl.BlockSpec((_BN, _D), lambda i: (i, 0)),
                  pl.BlockSpec((_BN, _D), lambda i: (i, 0)),
                  pl.BlockSpec((_BN, _D), lambda i: (i + nb, 0)),
                  _const_spec((_D, _D)), _const_spec((_D, _D)),
                  _const_spec((1, _D)), _const_spec((1, _D)),
                  _const_spec((1, _D)), _const_spec((_D, _D)),
                  _const_spec((1, _D))],
        out_specs=pl.BlockSpec((_BN, _D), lambda i: (i, 0)),
        out_shape=jax.ShapeDtypeStruct((_N, _D), jnp.float32),
    )(h, parts, parts, wh, wa, b1, g1, t1, w2, b2)


def _edgeupd_pl(e, g, we, ws, wd, b1, g1, t1, w2, b2):
    nb = _E // _BE

    def body(e_ref, ns_ref, nd_ref, we_, ws_, wd_, b1_, g1_, t1_, w2_, b2_, o_ref):
        h1 = (jnp.dot(e_ref[...].astype(jnp.bfloat16), we_[...], preferred_element_type=jnp.float32)
              + jnp.dot(ns_ref[...].astype(jnp.bfloat16), ws_[...], preferred_element_type=jnp.float32)
              + jnp.dot(nd_ref[...].astype(jnp.bfloat16), wd_[...], preferred_element_type=jnp.float32)
              + b1_[...])
        h1 = _ln_silu(h1, g1_[...], t1_[...])
        o_ref[...] = (jnp.dot(h1.astype(jnp.bfloat16), w2_[...], preferred_element_type=jnp.float32)
                      + b2_[...] + e_ref[...])

    return pl.pallas_call(
        body,
        grid=(nb,),
        in_specs=[pl.BlockSpec((_BE, _D), lambda i: (i, 0)),
                  pl.BlockSpec((_BE, _D), lambda i: (i, 0)),
                  pl.BlockSpec((_BE, _D), lambda i: (i + nb, 0)),
                  _const_spec((_D, _D)), _const_spec((_D, _D)),
                  _const_spec((_D, _D)), _const_spec((1, _D)),
                  _const_spec((1, _D)), _const_spec((1, _D)),
                  _const_spec((_D, _D)), _const_spec((1, _D))],
        out_specs=pl.BlockSpec((_BE, _D), lambda i: (i, 0)),
        out_shape=jax.ShapeDtypeStruct((_E, _D), jnp.float32),
    )(e, g, g, we, ws, wd, b1, g1, t1, w2, b2)


def _sc_gather(h, idx2):
    b = idx2.shape[0]
    per_w = b // _NW
    nch = per_w // _GCH
    mesh = plsc.VectorSubcoreMesh(core_axis_name="c", subcore_axis_name="s")

    @functools.partial(
        pl.kernel, mesh=mesh,
        out_type=jax.ShapeDtypeStruct((b, _D), jnp.float32),
        scratch_types=[pltpu.VMEM((per_w,), jnp.int32),
                       pltpu.VMEM((_GCH, _D), jnp.float32),
                       pltpu.VMEM((_GCH, _D), jnp.float32),
                       pltpu.SemaphoreType.DMA, pltpu.SemaphoreType.DMA,
                       pltpu.SemaphoreType.DMA, pltpu.SemaphoreType.DMA],
    )
    def k(h_hbm, idx_hbm, out_hbm, idx_v, r0, r1, g0, g1, w0, w1):
        wid = lax.axis_index("s") * 2 + lax.axis_index("c")
        base = wid * per_w
        pltpu.sync_copy(idx_hbm.at[pl.ds(base, per_w)], idx_v)
        rows = (r0, r1)
        gsem = (g0, g1)
        wsem = (w0, w1)

        def outer(j, carry):
            for t in range(2):
                i = j * 2 + t

                @pl.when(j > 0)
                def _():
                    pltpu.make_async_copy(
                        rows[t], out_hbm.at[pl.ds(base + i * _GCH, _GCH)],
                        wsem[t]).wait()

                pltpu.async_copy(
                    h_hbm.at[idx_v.at[pl.ds(i * _GCH, _GCH)]], rows[t],
                    gsem[t])
            for t in range(2):
                i = j * 2 + t
                pltpu.make_async_copy(
                    h_hbm.at[idx_v.at[pl.ds(i * _GCH, _GCH)]], rows[t],
                    gsem[t]).wait()
                pltpu.async_copy(rows[t],
                                 out_hbm.at[pl.ds(base + i * _GCH, _GCH)],
                                 wsem[t])
            return carry

        lax.fori_loop(0, nch // 2, outer, 0)
        for t in range(2):
            pltpu.make_async_copy(
                rows[t], out_hbm.at[pl.ds(base, _GCH)], wsem[t]).wait()

    return k(h, idx2)


def _sc_scatter(m, dst, zer):
    per_w = _E // _NW
    nch = per_w // _SCH
    stripe = 624
    tail = _N - 16 * stripe
    mesh = plsc.VectorSubcoreMesh(core_axis_name="c", subcore_axis_name="s")

    @functools.partial(
        pl.kernel, mesh=mesh,
        out_type=jax.ShapeDtypeStruct((2 * _N, _D), jnp.float32),
        scratch_types=[pltpu.VMEM((_SCH,), jnp.int32),
                       pltpu.VMEM((_SCH,), jnp.int32),
                       pltpu.VMEM((_SCH, _D), jnp.float32),
                       pltpu.VMEM((_SCH, _D), jnp.float32),
                       pltpu.VMEM_SHARED((_N, _D), jnp.float32),
                       pltpu.SemaphoreType.DMA, pltpu.SemaphoreType.DMA,
                       pltpu.SemaphoreType.DMA, pltpu.SemaphoreType.DMA],
    )
    def k(m_hbm, dst_hbm, zer_hbm, out_hbm, i0, i1, r0, r1, acc,
          mi0, mi1, mr0, mr1):
        cid = lax.axis_index("c")
        sid = lax.axis_index("s")
        wid = sid * 2 + cid
        pltpu.sync_copy(zer_hbm.at[pl.ds(sid * stripe, stripe)],
                        acc.at[pl.ds(sid * stripe, stripe)])
        @pl.when(sid == 15)
        def _():
            pltpu.sync_copy(zer_hbm.at[pl.ds(16 * stripe, tail)],
                            acc.at[pl.ds(16 * stripe, tail)])
        plsc.subcore_barrier()

        base = wid * per_w
        idx_b = (i0, i1)
        rows_b = (r0, r1)
        isem = (mi0, mi1)
        msem = (mr0, mr1)

        lead = nch % 2
        if lead:
            pltpu.sync_copy(dst_hbm.at[pl.ds(base, _SCH)], i0)
            pltpu.sync_copy(m_hbm.at[pl.ds(base, _SCH)], r0)
            pltpu.sync_copy(r0, acc.at[i0], add=True)

        def outer(j, carry):
            for t in range(2):
                i = lead + j * 2 + t
                off = base + i * _SCH
                pltpu.async_copy(dst_hbm.at[pl.ds(off, _SCH)], idx_b[t],
                                 isem[t])
                pltpu.async_copy(m_hbm.at[pl.ds(off, _SCH)], rows_b[t],
                                 msem[t])
            for t in range(2):
                i = lead + j * 2 + t
                off = base + i * _SCH
                pltpu.make_async_copy(dst_hbm.at[pl.ds(off, _SCH)], idx_b[t],
                                      isem[t]).wait()
                pltpu.make_async_copy(m_hbm.at[pl.ds(off, _SCH)], rows_b[t],
                                      msem[t]).wait()
                pltpu.sync_copy(rows_b[t], acc.at[idx_b[t]], add=True)
            return carry

        lax.fori_loop(0, (nch - lead) // 2, outer, 0)
        plsc.subcore_barrier()
        pltpu.sync_copy(
            acc.at[pl.ds(sid * stripe, stripe)],
            out_hbm.at[pl.ds(cid * _N + sid * stripe, stripe)])
        @pl.when(sid == 15)
        def _():
            pltpu.sync_copy(
                acc.at[pl.ds(16 * stripe, tail)],
                out_hbm.at[pl.ds(cid * _N + 16 * stripe, tail)])

    return k(m, dst, zer)


def _pack_layer(lp):
    mp, gp, np_, ep = lp["message"], lp["gate"], lp["node"], lp["edge"]
    w1 = jnp.concatenate([mp["l1"]["W"], gp["l1"]["W"]],
                         axis=1).astype(jnp.bfloat16)
    msg = dict(
        wd=w1[:_D], ws=w1[_D:2 * _D], we=w1[2 * _D:],
        b1=jnp.concatenate([mp["l1"]["b"], gp["l1"]["b"]]).reshape(1, 2 * _D),
        g1=jnp.concatenate([mp["ln_g"], gp["ln_g"]]).reshape(1, 2 * _D),
        t1=jnp.concatenate([mp["ln_b"], gp["ln_b"]]).reshape(1, 2 * _D),
        w2m=mp["l2"]["W"].astype(jnp.bfloat16), b2m=mp["l2"]["b"].reshape(1, _D),
        w2g=jnp.broadcast_to(gp["l2"]["W"], (_D, _D)).astype(jnp.bfloat16),
        b2g=jnp.broadcast_to(gp["l2"]["b"].reshape(1, 1), (1, _D)),
    )
    node = dict(
        wh=np_["l1"]["W"][:_D].astype(jnp.bfloat16),
        wa=np_["l1"]["W"][_D:].astype(jnp.bfloat16),
        b1=np_["l1"]["b"].reshape(1, _D),
        g1=np_["ln_g"].reshape(1, _D), t1=np_["ln_b"].reshape(1, _D),
        w2=np_["l2"]["W"].astype(jnp.bfloat16), b2=np_["l2"]["b"].reshape(1, _D),
    )
    edge = dict(
        we=ep["l1"]["W"][:_D].astype(jnp.bfloat16),
        ws=ep["l1"]["W"][_D:2 * _D].astype(jnp.bfloat16),
        wd=ep["l1"]["W"][2 * _D:].astype(jnp.bfloat16),
        b1=ep["l1"]["b"].reshape(1, _D),
        g1=ep["ln_g"].reshape(1, _D), t1=ep["ln_b"].reshape(1, _D),
        w2=ep["l2"]["W"].astype(jnp.bfloat16), b2=ep["l2"]["b"].reshape(1, _D),
    )
    return msg, node, edge


def kernel(x, edge_index, edge_attr, params):
    src = edge_index[0]
    dst = edge_index[1]
    idx2 = jnp.concatenate([src, dst])
    zer = jnp.zeros((_N, _D), jnp.float32)

    h = _linear_pl(x, params["node_enc"]["W"].astype(jnp.bfloat16),
                   params["node_enc"]["b"], _BN)
    e = _linear_pl(edge_attr, params["edge_enc"]["W"].astype(jnp.bfloat16),
                   params["edge_enc"]["b"], _BE)

    g = _sc_gather(h, idx2)
    for lp in params["layers"]:
        msg, node, edge = _pack_layer(lp)
        m = _message_pl(g, e, **msg)
        parts = _sc_scatter(m, dst, zer)
        h = _node_pl(h, parts, **node)
        g = _sc_gather(h, idx2)
        e = _edgeupd_pl(e, g, **edge)

    x_out = _linear_pl(h, params["node_dec"]["W"].astype(jnp.bfloat16),
                       params["node_dec"]["b"], _BN)
    e_out = _linear_pl(e, params["edge_dec"]["W"].astype(jnp.bfloat16),
                       params["edge_dec"]["b"], _BE)
    return (x_out, e_out)


# speedup vs baseline: 1.1468x; 1.1468x over previous
import functools

import jax
import jax.numpy as jnp
from jax import lax
from jax.experimental import pallas as pl
from jax.experimental.pallas import tpu as pltpu
from jax.experimental.pallas import tpu_sc as plsc

_N = 10000
_E = 320000
_D = 128
_LN_EPS = 1e-5

_BE = 2000
_BN = 2000
_NW = 32
_GCH = 80
_SCH = 80


def _sig(x):
    return 0.5 * jnp.tanh(0.5 * x) + 0.5


def _ln_silu(h, g, b):
    mu = jnp.mean(h, axis=-1, keepdims=True)
    c = h - mu
    var = jnp.mean(c * c, axis=-1, keepdims=True)
    hn = c * lax.rsqrt(var + _LN_EPS) * g + b
    return hn * _sig(hn)


def _const_spec(shape):
    return pl.BlockSpec(shape, lambda i: tuple(0 for _ in shape))


def _linear_pl(x, w, b, block):
    n, k = x.shape
    m = w.shape[1]
    def body(x_ref, w_ref, b_ref, o_ref):
        o_ref[...] = (
            jnp.dot(x_ref[...], w_ref[...], preferred_element_type=jnp.float32)
            + b_ref[...])
    return pl.pallas_call(
        body,
        grid=(n // block,),
        in_specs=[pl.BlockSpec((block, k), lambda i: (i, 0)),
                  _const_spec((k, m)),
                  _const_spec((1, m))],
        out_specs=pl.BlockSpec((block, m), lambda i: (i, 0)),
        out_shape=jax.ShapeDtypeStruct((n, m), jnp.float32),
    )(x, w, b.reshape(1, m))


def _message_pl(g, e, wd, ws, we, b1, g1, t1, w2m, b2m, w2g, b2g):
    nb = _E // _BE

    def body(hd_ref, hs_ref, e_ref, wd_, ws_, we_, b1_, g1_, t1_,
             w2m_, b2m_, w2g_, b2g_, o_ref):
        h1 = (jnp.dot(hd_ref[...], wd_[...], preferred_element_type=jnp.float32)
              + jnp.dot(hs_ref[...], ws_[...], preferred_element_type=jnp.float32)
              + jnp.dot(e_ref[...], we_[...], preferred_element_type=jnp.float32)
              + b1_[...])
        hm = _ln_silu(h1[:, :_D], g1_[:, :_D], t1_[:, :_D])
        hg = _ln_silu(h1[:, _D:], g1_[:, _D:], t1_[:, _D:])
        msg = jnp.dot(hm, w2m_[...], preferred_element_type=jnp.float32) + b2m_[...]
        gl = jnp.dot(hg, w2g_[...], preferred_element_type=jnp.float32) + b2g_[...]
        o_ref[...] = _sig(gl) * msg

    return pl.pallas_call(
        body,
        grid=(nb,),
        in_specs=[pl.BlockSpec((_BE, _D), lambda i: (i + nb, 0)),
                  pl.BlockSpec((_BE, _D), lambda i: (i, 0)),
                  pl.BlockSpec((_BE, _D), lambda i: (i, 0)),
                  _const_spec((_D, 2 * _D)), _const_spec((_D, 2 * _D)),
                  _const_spec((_D, 2 * _D)), _const_spec((1, 2 * _D)),
                  _const_spec((1, 2 * _D)), _const_spec((1, 2 * _D)),
                  _const_spec((_D, _D)), _const_spec((1, _D)),
                  _const_spec((_D, _D)), _const_spec((1, _D))],
        out_specs=pl.BlockSpec((_BE, _D), lambda i: (i, 0)),
        out_shape=jax.ShapeDtypeStruct((_E, _D), jnp.float32),
    )(g, g, e, wd, ws, we, b1, g1, t1, w2m, b2m, w2g, b2g)


def _node_pl(h, parts, wh, wa, b1, g1, t1, w2, b2):
    nb = _N // _BN

    def body(h_ref, p0_ref, p1_ref, wh_, wa_, b1_, g1_, t1_, w2_, b2_, o_ref):
        aggr = p0_ref[...] + p1_ref[...]
        h1 = (jnp.dot(h_ref[...], wh_[...], preferred_element_type=jnp.float32)
              + jnp.dot(aggr, wa_[...], preferred_element_type=jnp.float32)
              + b1_[...])
        h1 = _ln_silu(h1, g1_[...], t1_[...])
        o_ref[...] = (jnp.dot(h1, w2_[...], preferred_element_type=jnp.float32)
                      + b2_[...] + h_ref[...])

    return pl.pallas_call(
        body,
        grid=(nb,),
        in_specs=[pl.BlockSpec((_BN, _D), lambda i: (i, 0)),
                  pl.BlockSpec((_BN, _D), lambda i: (i, 0)),
                  pl.BlockSpec((_BN, _D), lambda i: (i + nb, 0)),
                  _const_spec((_D, _D)), _const_spec((_D, _D)),
                  _const_spec((1, _D)), _const_spec((1, _D)),
                  _const_spec((1, _D)), _const_spec((_D, _D)),
                  _const_spec((1, _D))],
        out_specs=pl.BlockSpec((_BN, _D), lambda i: (i, 0)),
        out_shape=jax.ShapeDtypeStruct((_N, _D), jnp.float32),
    )(h, parts, parts, wh, wa, b1, g1, t1, w2, b2)


def _edgeupd_pl(e, g, we, ws, wd, b1, g1, t1, w2, b2):
    nb = _E // _BE

    def body(e_ref, ns_ref, nd_ref, we_, ws_, wd_, b1_, g1_, t1_, w2_, b2_, o_ref):
        h1 = (jnp.dot(e_ref[...], we_[...], preferred_element_type=jnp.float32)
              + jnp.dot(ns_ref[...], ws_[...], preferred_element_type=jnp.float32)
              + jnp.dot(nd_ref[...], wd_[...], preferred_element_type=jnp.float32)
              + b1_[...])
        h1 = _ln_silu(h1, g1_[...], t1_[...])
        o_ref[...] = (jnp.dot(h1, w2_[...], preferred_element_type=jnp.float32)
                      + b2_[...] + e_ref[...])

    return pl.pallas_call(
        body,
        grid=(nb,),
        in_specs=[pl.BlockSpec((_BE, _D), lambda i: (i, 0)),
                  pl.BlockSpec((_BE, _D), lambda i: (i, 0)),
                  pl.BlockSpec((_BE, _D), lambda i: (i + nb, 0)),
                  _const_spec((_D, _D)), _const_spec((_D, _D)),
                  _const_spec((_D, _D)), _const_spec((1, _D)),
                  _const_spec((1, _D)), _const_spec((1, _D)),
                  _const_spec((_D, _D)), _const_spec((1, _D))],
        out_specs=pl.BlockSpec((_BE, _D), lambda i: (i, 0)),
        out_shape=jax.ShapeDtypeStruct((_E, _D), jnp.float32),
    )(e, g, g, we, ws, wd, b1, g1, t1, w2, b2)


def _sc_gather(h, idx2):
    b = idx2.shape[0]
    per_w = b // _NW
    nch = per_w // _GCH
    mesh = plsc.VectorSubcoreMesh(core_axis_name="c", subcore_axis_name="s")

    @functools.partial(
        pl.kernel, mesh=mesh,
        out_type=jax.ShapeDtypeStruct((b, _D), jnp.float32),
        scratch_types=[pltpu.VMEM((per_w,), jnp.int32),
                       pltpu.VMEM((_GCH, _D), jnp.float32),
                       pltpu.VMEM((_GCH, _D), jnp.float32),
                       pltpu.SemaphoreType.DMA, pltpu.SemaphoreType.DMA,
                       pltpu.SemaphoreType.DMA, pltpu.SemaphoreType.DMA],
    )
    def k(h_hbm, idx_hbm, out_hbm, idx_v, r0, r1, g0, g1, w0, w1):
        wid = lax.axis_index("s") * 2 + lax.axis_index("c")
        base = wid * per_w
        pltpu.sync_copy(idx_hbm.at[pl.ds(base, per_w)], idx_v)
        rows = (r0, r1)
        gsem = (g0, g1)
        wsem = (w0, w1)

        def outer(j, carry):
            for t in range(2):
                i = j * 2 + t

                @pl.when(j > 0)
                def _():
                    pltpu.make_async_copy(
                        rows[t], out_hbm.at[pl.ds(base + i * _GCH, _GCH)],
                        wsem[t]).wait()

                pltpu.async_copy(
                    h_hbm.at[idx_v.at[pl.ds(i * _GCH, _GCH)]], rows[t],
                    gsem[t])
            for t in range(2):
                i = j * 2 + t
                pltpu.make_async_copy(
                    h_hbm.at[idx_v.at[pl.ds(i * _GCH, _GCH)]], rows[t],
                    gsem[t]).wait()
                pltpu.async_copy(rows[t],
                                 out_hbm.at[pl.ds(base + i * _GCH, _GCH)],
                                 wsem[t])
            return carry

        lax.fori_loop(0, nch // 2, outer, 0)
        for t in range(2):
            pltpu.make_async_copy(
                rows[t], out_hbm.at[pl.ds(base, _GCH)], wsem[t]).wait()

    return k(h, idx2)


def _sc_scatter(m, dst, zer):
    per_w = _E // _NW
    nch = per_w // _SCH
    stripe = 624
    tail = _N - 16 * stripe
    mesh = plsc.VectorSubcoreMesh(core_axis_name="c", subcore_axis_name="s")

    @functools.partial(
        pl.kernel, mesh=mesh,
        out_type=jax.ShapeDtypeStruct((2 * _N, _D), jnp.float32),
        scratch_types=[pltpu.VMEM((_SCH,), jnp.int32),
                       pltpu.VMEM((_SCH,), jnp.int32),
                       pltpu.VMEM((_SCH, _D), jnp.float32),
                       pltpu.VMEM((_SCH, _D), jnp.float32),
                       pltpu.VMEM_SHARED((_N, _D), jnp.float32),
                       pltpu.SemaphoreType.DMA, pltpu.SemaphoreType.DMA,
                       pltpu.SemaphoreType.DMA, pltpu.SemaphoreType.DMA],
    )
    def k(m_hbm, dst_hbm, zer_hbm, out_hbm, i0, i1, r0, r1, acc,
          mi0, mi1, mr0, mr1):
        cid = lax.axis_index("c")
        sid = lax.axis_index("s")
        wid = sid * 2 + cid
        pltpu.sync_copy(zer_hbm.at[pl.ds(sid * stripe, stripe)],
                        acc.at[pl.ds(sid * stripe, stripe)])
        @pl.when(sid == 15)
        def _():
            pltpu.sync_copy(zer_hbm.at[pl.ds(16 * stripe, tail)],
                            acc.at[pl.ds(16 * stripe, tail)])
        plsc.subcore_barrier()

        base = wid * per_w
        idx_b = (i0, i1)
        rows_b = (r0, r1)
        isem = (mi0, mi1)
        msem = (mr0, mr1)

        lead = nch % 2
        if lead:
            pltpu.sync_copy(dst_hbm.at[pl.ds(base, _SCH)], i0)
            pltpu.sync_copy(m_hbm.at[pl.ds(base, _SCH)], r0)
            pltpu.sync_copy(r0, acc.at[i0], add=True)

        def outer(j, carry):
            for t in range(2):
                i = lead + j * 2 + t
                off = base + i * _SCH
                pltpu.async_copy(dst_hbm.at[pl.ds(off, _SCH)], idx_b[t],
                                 isem[t])
                pltpu.async_copy(m_hbm.at[pl.ds(off, _SCH)], rows_b[t],
                                 msem[t])
            for t in range(2):
                i = lead + j * 2 + t
                off = base + i * _SCH
                pltpu.make_async_copy(dst_hbm.at[pl.ds(off, _SCH)], idx_b[t],
                                      isem[t]).wait()
                pltpu.make_async_copy(m_hbm.at[pl.ds(off, _SCH)], rows_b[t],
                                      msem[t]).wait()
                pltpu.sync_copy(rows_b[t], acc.at[idx_b[t]], add=True)
            return carry

        lax.fori_loop(0, (nch - lead) // 2, outer, 0)
        plsc.subcore_barrier()
        pltpu.sync_copy(
            acc.at[pl.ds(sid * stripe, stripe)],
            out_hbm.at[pl.ds(cid * _N + sid * stripe, stripe)])
        @pl.when(sid == 15)
        def _():
            pltpu.sync_copy(
                acc.at[pl.ds(16 * stripe, tail)],
                out_hbm.at[pl.ds(cid * _N + 16 * stripe, tail)])

    return k(m, dst, zer)


def _pack_layer(lp):
    mp, gp, np_, ep = lp["message"], lp["gate"], lp["node"], lp["edge"]
    w1 = jnp.concatenate([mp["l1"]["W"], gp["l1"]["W"]], axis=1)
    msg = dict(
        wd=w1[:_D], ws=w1[_D:2 * _D], we=w1[2 * _D:],
        b1=jnp.concatenate([mp["l1"]["b"], gp["l1"]["b"]]).reshape(1, 2 * _D),
        g1=jnp.concatenate([mp["ln_g"], gp["ln_g"]]).reshape(1, 2 * _D),
        t1=jnp.concatenate([mp["ln_b"], gp["ln_b"]]).reshape(1, 2 * _D),
        w2m=mp["l2"]["W"], b2m=mp["l2"]["b"].reshape(1, _D),
        w2g=jnp.broadcast_to(gp["l2"]["W"], (_D, _D)),
        b2g=jnp.broadcast_to(gp["l2"]["b"].reshape(1, 1), (1, _D)),
    )
    node = dict(
        wh=np_["l1"]["W"][:_D],
        wa=np_["l1"]["W"][_D:],
        b1=np_["l1"]["b"].reshape(1, _D),
        g1=np_["ln_g"].reshape(1, _D), t1=np_["ln_b"].reshape(1, _D),
        w2=np_["l2"]["W"], b2=np_["l2"]["b"].reshape(1, _D),
    )
    edge = dict(
        we=ep["l1"]["W"][:_D],
        ws=ep["l1"]["W"][_D:2 * _D],
        wd=ep["l1"]["W"][2 * _D:],
        b1=ep["l1"]["b"].reshape(1, _D),
        g1=ep["ln_g"].reshape(1, _D), t1=ep["ln_b"].reshape(1, _D),
        w2=ep["l2"]["W"], b2=ep["l2"]["b"].reshape(1, _D),
    )
    return msg, node, edge


def kernel(x, edge_index, edge_attr, params):
    src = edge_index[0]
    dst = edge_index[1]
    idx2 = jnp.concatenate([src, dst])
    zer = jnp.zeros((_N, _D), jnp.float32)

    h = _linear_pl(x, params["node_enc"]["W"],
                   params["node_enc"]["b"], _BN)
    e = _linear_pl(edge_attr, params["edge_enc"]["W"],
                   params["edge_enc"]["b"], _BE)

    g = _sc_gather(h, idx2)
    for lp in params["layers"]:
        msg, node, edge = _pack_layer(lp)
        m = _message_pl(g, e, **msg)
        parts = _sc_scatter(m, dst, zer)
        h = _node_pl(h, parts, **node)
        g = _sc_gather(h, idx2)
        e = _edgeupd_pl(e, g, **edge)

    x_out = _linear_pl(h, params["node_dec"]["W"],
                       params["node_dec"]["b"], _BN)
    e_out = _linear_pl(e, params["edge_dec"]["W"],
                       params["edge_dec"]["b"], _BE)
    return (x_out, e_out)


# speedup vs baseline: 1.2238x; 1.0672x over previous
import functools

import jax
import jax.numpy as jnp
from jax import lax
from jax.experimental import pallas as pl
from jax.experimental.pallas import tpu as pltpu
from jax.experimental.pallas import tpu_sc as plsc

_N = 10000
_E = 320000
_EH = _E // 2
_D = 128
_LN_EPS = 1e-5

_BE = 2000
_BN = 2000
_NW = 32
_GCH = 80
_SCH = 40


def _sig(x):
    return 0.5 * jnp.tanh(0.5 * x) + 0.5


def _ln_silu(h, g, b):
    mu = jnp.mean(h, axis=-1, keepdims=True)
    c = h - mu
    var = jnp.mean(c * c, axis=-1, keepdims=True)
    hn = c * lax.rsqrt(var + _LN_EPS) * g + b
    return hn * _sig(hn)


def _const_spec(shape):
    return pl.BlockSpec(shape, lambda i: tuple(0 for _ in shape))


def _linear_pl(x, w, b, block):
    n, k = x.shape
    m = w.shape[1]

    def body(x_ref, w_ref, b_ref, o_ref):
        o_ref[...] = (
            jnp.dot(x_ref[...], w_ref[...], preferred_element_type=jnp.float32)
            + b_ref[...])

    return pl.pallas_call(
        body,
        grid=(n // block,),
        in_specs=[pl.BlockSpec((block, k), lambda i: (i, 0)),
                  _const_spec((k, m)),
                  _const_spec((1, m))],
        out_specs=pl.BlockSpec((block, m), lambda i: (i, 0)),
        out_shape=jax.ShapeDtypeStruct((n, m), jnp.float32),
    )(x, w, b.reshape(1, m))


def _message_pl(g, e, wd, ws, we, b1, g1, t1, w2m, b2m, w2g, b2g):
    nb = _EH // _BE

    def body(hd_ref, hs_ref, e_ref, wd_, ws_, we_, b1_, g1_, t1_,
             w2m_, b2m_, w2g_, b2g_, o_ref):
        h1 = (jnp.dot(hd_ref[...], wd_[...], preferred_element_type=jnp.float32)
              + jnp.dot(hs_ref[...], ws_[...], preferred_element_type=jnp.float32)
              + jnp.dot(e_ref[...], we_[...], preferred_element_type=jnp.float32)
              + b1_[...])
        hm = _ln_silu(h1[:, :_D], g1_[:, :_D], t1_[:, :_D])
        hg = _ln_silu(h1[:, _D:], g1_[:, _D:], t1_[:, _D:])
        msg = jnp.dot(hm, w2m_[...], preferred_element_type=jnp.float32) + b2m_[...]
        gl = jnp.dot(hg, w2g_[...], preferred_element_type=jnp.float32) + b2g_[...]
        o_ref[...] = _sig(gl) * msg

    return pl.pallas_call(
        body,
        grid=(nb,),
        in_specs=[pl.BlockSpec((_BE, _D), lambda i: (i + nb, 0)),
                  pl.BlockSpec((_BE, _D), lambda i: (i, 0)),
                  pl.BlockSpec((_BE, _D), lambda i: (i, 0)),
                  _const_spec((_D, 2 * _D)), _const_spec((_D, 2 * _D)),
                  _const_spec((_D, 2 * _D)), _const_spec((1, 2 * _D)),
                  _const_spec((1, 2 * _D)), _const_spec((1, 2 * _D)),
                  _const_spec((_D, _D)), _const_spec((1, _D)),
                  _const_spec((_D, _D)), _const_spec((1, _D))],
        out_specs=pl.BlockSpec((_BE, _D), lambda i: (i, 0)),
        out_shape=jax.ShapeDtypeStruct((_EH, _D), jnp.float32),
    )(g, g, e, wd, ws, we, b1, g1, t1, w2m, b2m, w2g, b2g)


def _node_pl(h, pa, pb, wh, wa, b1, g1, t1, w2, b2):
    nb = _N // _BN

    def body(h_ref, pa0_ref, pa1_ref, pb0_ref, pb1_ref,
             wh_, wa_, b1_, g1_, t1_, w2_, b2_, o_ref):
        aggr = (pa0_ref[...] + pa1_ref[...]) + (pb0_ref[...] + pb1_ref[...])
        h1 = (jnp.dot(h_ref[...], wh_[...], preferred_element_type=jnp.float32)
              + jnp.dot(aggr, wa_[...], preferred_element_type=jnp.float32)
              + b1_[...])
        h1 = _ln_silu(h1, g1_[...], t1_[...])
        o_ref[...] = (jnp.dot(h1, w2_[...], preferred_element_type=jnp.float32)
                      + b2_[...] + h_ref[...])

    return pl.pallas_call(
        body,
        grid=(nb,),
        in_specs=[pl.BlockSpec((_BN, _D), lambda i: (i, 0)),
                  pl.BlockSpec((_BN, _D), lambda i: (i, 0)),
                  pl.BlockSpec((_BN, _D), lambda i: (i + nb, 0)),
                  pl.BlockSpec((_BN, _D), lambda i: (i, 0)),
                  pl.BlockSpec((_BN, _D), lambda i: (i + nb, 0)),
                  _const_spec((_D, _D)), _const_spec((_D, _D)),
                  _const_spec((1, _D)), _const_spec((1, _D)),
                  _const_spec((1, _D)), _const_spec((_D, _D)),
                  _const_spec((1, _D))],
        out_specs=pl.BlockSpec((_BN, _D), lambda i: (i, 0)),
        out_shape=jax.ShapeDtypeStruct((_N, _D), jnp.float32),
    )(h, pa, pa, pb, pb, wh, wa, b1, g1, t1, w2, b2)


def _edgeupd_pl(e, g, we, ws, wd, b1, g1, t1, w2, b2):
    nb = _EH // _BE

    def body(e_ref, ns_ref, nd_ref, we_, ws_, wd_, b1_, g1_, t1_, w2_, b2_,
             o_ref):
        h1 = (jnp.dot(e_ref[...], we_[...], preferred_element_type=jnp.float32)
              + jnp.dot(ns_ref[...], ws_[...], preferred_element_type=jnp.float32)
              + jnp.dot(nd_ref[...], wd_[...], preferred_element_type=jnp.float32)
              + b1_[...])
        h1 = _ln_silu(h1, g1_[...], t1_[...])
        o_ref[...] = (jnp.dot(h1, w2_[...], preferred_element_type=jnp.float32)
                      + b2_[...] + e_ref[...])

    return pl.pallas_call(
        body,
        grid=(nb,),
        in_specs=[pl.BlockSpec((_BE, _D), lambda i: (i, 0)),
                  pl.BlockSpec((_BE, _D), lambda i: (i, 0)),
                  pl.BlockSpec((_BE, _D), lambda i: (i + nb, 0)),
                  _const_spec((_D, _D)), _const_spec((_D, _D)),
                  _const_spec((_D, _D)), _const_spec((1, _D)),
                  _const_spec((1, _D)), _const_spec((1, _D)),
                  _const_spec((_D, _D)), _const_spec((1, _D))],
        out_specs=pl.BlockSpec((_BE, _D), lambda i: (i, 0)),
        out_shape=jax.ShapeDtypeStruct((_EH, _D), jnp.float32),
    )(e, g, g, we, ws, wd, b1, g1, t1, w2, b2)


def _sc_gather(h, idx):
    b = idx.shape[0]
    per_w = b // _NW
    nch = per_w // _GCH
    lead = nch % 2
    mesh = plsc.VectorSubcoreMesh(core_axis_name="c", subcore_axis_name="s")

    @functools.partial(
        pl.kernel, mesh=mesh,
        out_type=jax.ShapeDtypeStruct((b, _D), jnp.float32),
        scratch_types=[pltpu.VMEM((per_w,), jnp.int32),
                       pltpu.VMEM((_GCH, _D), jnp.float32),
                       pltpu.VMEM((_GCH, _D), jnp.float32),
                       pltpu.SemaphoreType.DMA, pltpu.SemaphoreType.DMA,
                       pltpu.SemaphoreType.DMA, pltpu.SemaphoreType.DMA],
    )
    def k(h_hbm, idx_hbm, out_hbm, idx_v, r0, r1, g0, g1, w0, w1):
        wid = lax.axis_index("s") * 2 + lax.axis_index("c")
        base = wid * per_w
        pltpu.sync_copy(idx_hbm.at[pl.ds(base, per_w)], idx_v)
        rows = (r0, r1)
        gsem = (g0, g1)
        wsem = (w0, w1)

        if lead:
            pltpu.async_copy(h_hbm.at[idx_v.at[pl.ds(0, _GCH)]], r0, g0).wait()
            pltpu.sync_copy(r0, out_hbm.at[pl.ds(base, _GCH)])

        def outer(j, carry):
            for t in range(2):
                i = lead + j * 2 + t

                @pl.when(j > 0)
                def _():
                    pltpu.make_async_copy(
                        rows[t], out_hbm.at[pl.ds(base + i * _GCH, _GCH)],
                        wsem[t]).wait()

                pltpu.async_copy(
                    h_hbm.at[idx_v.at[pl.ds(i * _GCH, _GCH)]], rows[t],
                    gsem[t])
            for t in range(2):
                i = lead + j * 2 + t
                pltpu.make_async_copy(
                    h_hbm.at[idx_v.at[pl.ds(i * _GCH, _GCH)]], rows[t],
                    gsem[t]).wait()
                pltpu.async_copy(rows[t],
                                 out_hbm.at[pl.ds(base + i * _GCH, _GCH)],
                                 wsem[t])
            return carry

        lax.fori_loop(0, (nch - lead) // 2, outer, 0)
        for t in range(2):
            pltpu.make_async_copy(
                rows[t], out_hbm.at[pl.ds(base, _GCH)], wsem[t]).wait()

    return k(h, idx)


def _sc_scatter(m, dsth, zer):
    per_w = _EH // _NW
    nch = per_w // _SCH
    stripe = 624
    tail = _N - 16 * stripe
    mesh = plsc.VectorSubcoreMesh(core_axis_name="c", subcore_axis_name="s")

    @functools.partial(
        pl.kernel, mesh=mesh,
        out_type=jax.ShapeDtypeStruct((2 * _N, _D), jnp.float32),
        scratch_types=[pltpu.VMEM((_SCH,), jnp.int32),
                       pltpu.VMEM((_SCH,), jnp.int32),
                       pltpu.VMEM((_SCH, _D), jnp.float32),
                       pltpu.VMEM((_SCH, _D), jnp.float32),
                       pltpu.VMEM_SHARED((_N, _D), jnp.float32),
                       pltpu.SemaphoreType.DMA, pltpu.SemaphoreType.DMA,
                       pltpu.SemaphoreType.DMA, pltpu.SemaphoreType.DMA],
    )
    def k(m_hbm, dst_hbm, zer_hbm, out_hbm, i0, i1, r0, r1, acc,
          mi0, mi1, mr0, mr1):
        cid = lax.axis_index("c")
        sid = lax.axis_index("s")
        wid = sid * 2 + cid
        pltpu.sync_copy(zer_hbm.at[pl.ds(sid * stripe, stripe)],
                        acc.at[pl.ds(sid * stripe, stripe)])

        @pl.when(sid == 15)
        def _():
            pltpu.sync_copy(zer_hbm.at[pl.ds(16 * stripe, tail)],
                            acc.at[pl.ds(16 * stripe, tail)])

        plsc.subcore_barrier()

        base = wid * per_w
        idx_b = (i0, i1)
        rows_b = (r0, r1)
        isem = (mi0, mi1)
        msem = (mr0, mr1)

        lead = nch % 2
        if lead:
            pltpu.sync_copy(dst_hbm.at[pl.ds(base, _SCH)], i0)
            pltpu.sync_copy(m_hbm.at[pl.ds(base, _SCH)], r0)
            pltpu.sync_copy(r0, acc.at[i0], add=True)

        def outer(j, carry):
            for t in range(2):
                i = lead + j * 2 + t
                off = base + i * _SCH
                pltpu.async_copy(dst_hbm.at[pl.ds(off, _SCH)], idx_b[t],
                                 isem[t])
                pltpu.async_copy(m_hbm.at[pl.ds(off, _SCH)], rows_b[t],
                                 msem[t])
            for t in range(2):
                i = lead + j * 2 + t
                off = base + i * _SCH
                pltpu.make_async_copy(dst_hbm.at[pl.ds(off, _SCH)], idx_b[t],
                                      isem[t]).wait()
                pltpu.make_async_copy(m_hbm.at[pl.ds(off, _SCH)], rows_b[t],
                                      msem[t]).wait()
                pltpu.sync_copy(rows_b[t], acc.at[idx_b[t]], add=True)
            return carry

        lax.fori_loop(0, (nch - lead) // 2, outer, 0)
        plsc.subcore_barrier()
        pltpu.sync_copy(
            acc.at[pl.ds(sid * stripe, stripe)],
            out_hbm.at[pl.ds(cid * _N + sid * stripe, stripe)])

        @pl.when(sid == 15)
        def _():
            pltpu.sync_copy(
                acc.at[pl.ds(16 * stripe, tail)],
                out_hbm.at[pl.ds(cid * _N + 16 * stripe, tail)])

    return k(m, dsth, zer)


def _pack_layer(lp):
    mp, gp, np_, ep = lp["message"], lp["gate"], lp["node"], lp["edge"]
    w1 = jnp.concatenate([mp["l1"]["W"], gp["l1"]["W"]], axis=1)
    msg = dict(
        wd=w1[:_D], ws=w1[_D:2 * _D], we=w1[2 * _D:],
        b1=jnp.concatenate([mp["l1"]["b"], gp["l1"]["b"]]).reshape(1, 2 * _D),
        g1=jnp.concatenate([mp["ln_g"], gp["ln_g"]]).reshape(1, 2 * _D),
        t1=jnp.concatenate([mp["ln_b"], gp["ln_b"]]).reshape(1, 2 * _D),
        w2m=mp["l2"]["W"], b2m=mp["l2"]["b"].reshape(1, _D),
        w2g=jnp.broadcast_to(gp["l2"]["W"], (_D, _D)),
        b2g=jnp.broadcast_to(gp["l2"]["b"].reshape(1, 1), (1, _D)),
    )
    node = dict(
        wh=np_["l1"]["W"][:_D], wa=np_["l1"]["W"][_D:],
        b1=np_["l1"]["b"].reshape(1, _D),
        g1=np_["ln_g"].reshape(1, _D), t1=np_["ln_b"].reshape(1, _D),
        w2=np_["l2"]["W"], b2=np_["l2"]["b"].reshape(1, _D),
    )
    edge = dict(
        we=ep["l1"]["W"][:_D], ws=ep["l1"]["W"][_D:2 * _D],
        wd=ep["l1"]["W"][2 * _D:],
        b1=ep["l1"]["b"].reshape(1, _D),
        g1=ep["ln_g"].reshape(1, _D), t1=ep["ln_b"].reshape(1, _D),
        w2=ep["l2"]["W"], b2=ep["l2"]["b"].reshape(1, _D),
    )
    return msg, node, edge


def kernel(x, edge_index, edge_attr, params):
    src = edge_index[0]
    dst = edge_index[1]
    dst_a, dst_b = dst[:_EH], dst[_EH:]
    idx_a = jnp.concatenate([src[:_EH], dst_a])
    idx_b = jnp.concatenate([src[_EH:], dst_b])
    zer = jnp.zeros((_N, _D), jnp.float32)

    h = _linear_pl(x, params["node_enc"]["W"], params["node_enc"]["b"], _BN)
    e_a = _linear_pl(edge_attr[:_EH], params["edge_enc"]["W"],
                     params["edge_enc"]["b"], _BE)
    e_b = _linear_pl(edge_attr[_EH:], params["edge_enc"]["W"],
                     params["edge_enc"]["b"], _BE)

    g_a = _sc_gather(h, idx_a)
    g_b = _sc_gather(h, idx_b)
    for lp in params["layers"]:
        msg, node, edge = _pack_layer(lp)
        m_a = _message_pl(g_a, e_a, **msg)
        parts_a = _sc_scatter(m_a, dst_a, zer)
        m_b = _message_pl(g_b, e_b, **msg)
        parts_b = _sc_scatter(m_b, dst_b, zer)
        h = _node_pl(h, parts_a, parts_b, **node)
        g_a = _sc_gather(h, idx_a)
        e_a = _edgeupd_pl(e_a, g_a, **edge)
        g_b = _sc_gather(h, idx_b)
        e_b = _edgeupd_pl(e_b, g_b, **edge)

    x_out = _linear_pl(h, params["node_dec"]["W"], params["node_dec"]["b"], _BN)
    e_out_a = _linear_pl(e_a, params["edge_dec"]["W"],
                         params["edge_dec"]["b"], _BE)
    e_out_b = _linear_pl(e_b, params["edge_dec"]["W"],
                         params["edge_dec"]["b"], _BE)
    return (x_out, jnp.concatenate([e_out_a, e_out_b], axis=0))


# speedup vs baseline: 1.3275x; 1.0848x over previous
import functools

import jax
import jax.numpy as jnp
from jax import lax
from jax.experimental import pallas as pl
from jax.experimental.pallas import tpu as pltpu
from jax.experimental.pallas import tpu_sc as plsc

_N = 10000
_E = 320000
_EH = _E // 2
_D = 128
_LN_EPS = 1e-5

_BE = 4000
_BN = 2000
_NW = 32
_GCH = 80
_SCH = 40


def _sig(x):
    return 0.5 * jnp.tanh(0.5 * x) + 0.5


def _ln_silu(h, g, b):
    mu = jnp.mean(h, axis=-1, keepdims=True)
    c = h - mu
    var = jnp.mean(c * c, axis=-1, keepdims=True)
    hn = c * lax.rsqrt(var + _LN_EPS) * g + b
    return hn * _sig(hn)


def _const_spec(shape):
    return pl.BlockSpec(shape, lambda i: tuple(0 for _ in shape))


def _linear_pl(x, w, b, block):
    n, k = x.shape
    m = w.shape[1]

    def body(x_ref, w_ref, b_ref, o_ref):
        o_ref[...] = (
            jnp.dot(x_ref[...], w_ref[...], preferred_element_type=jnp.float32)
            + b_ref[...])

    return pl.pallas_call(
        body,
        grid=(n // block,),
        in_specs=[pl.BlockSpec((block, k), lambda i: (i, 0)),
                  _const_spec((k, m)),
                  _const_spec((1, m))],
        out_specs=pl.BlockSpec((block, m), lambda i: (i, 0)),
        out_shape=jax.ShapeDtypeStruct((n, m), jnp.float32),
    )(x, w, b.reshape(1, m))


def _message_pl(g, e, wd, ws, we, b1, g1, t1, w2m, b2m, w2g, b2g):
    nb = _EH // _BE

    def body(hd_ref, hs_ref, e_ref, wd_, ws_, we_, b1_, g1_, t1_,
             w2m_, b2m_, w2g_, b2g_, o_ref):
        h1 = (jnp.dot(hd_ref[...], wd_[...], preferred_element_type=jnp.float32)
              + jnp.dot(hs_ref[...], ws_[...], preferred_element_type=jnp.float32)
              + jnp.dot(e_ref[...], we_[...], preferred_element_type=jnp.float32)
              + b1_[...])
        hm = _ln_silu(h1[:, :_D], g1_[:, :_D], t1_[:, :_D])
        hg = _ln_silu(h1[:, _D:], g1_[:, _D:], t1_[:, _D:])
        msg = jnp.dot(hm, w2m_[...], preferred_element_type=jnp.float32) + b2m_[...]
        gl = jnp.dot(hg, w2g_[...], preferred_element_type=jnp.float32) + b2g_[...]
        o_ref[...] = _sig(gl) * msg

    return pl.pallas_call(
        body,
        grid=(nb,),
        in_specs=[pl.BlockSpec((_BE, _D), lambda i: (i + nb, 0)),
                  pl.BlockSpec((_BE, _D), lambda i: (i, 0)),
                  pl.BlockSpec((_BE, _D), lambda i: (i, 0)),
                  _const_spec((_D, 2 * _D)), _const_spec((_D, 2 * _D)),
                  _const_spec((_D, 2 * _D)), _const_spec((1, 2 * _D)),
                  _const_spec((1, 2 * _D)), _const_spec((1, 2 * _D)),
                  _const_spec((_D, _D)), _const_spec((1, _D)),
                  _const_spec((_D, _D)), _const_spec((1, _D))],
        out_specs=pl.BlockSpec((_BE, _D), lambda i: (i, 0)),
        out_shape=jax.ShapeDtypeStruct((_EH, _D), jnp.float32),
    )(g, g, e, wd, ws, we, b1, g1, t1, w2m, b2m, w2g, b2g)


def _node_pl(h, pa, pb, wh, wa, b1, g1, t1, w2, b2):
    nb = _N // _BN

    def body(h_ref, pa0_ref, pa1_ref, pb0_ref, pb1_ref,
             wh_, wa_, b1_, g1_, t1_, w2_, b2_, o_ref):
        aggr = (pa0_ref[...] + pa1_ref[...]) + (pb0_ref[...] + pb1_ref[...])
        h1 = (jnp.dot(h_ref[...], wh_[...], preferred_element_type=jnp.float32)
              + jnp.dot(aggr, wa_[...], preferred_element_type=jnp.float32)
              + b1_[...])
        h1 = _ln_silu(h1, g1_[...], t1_[...])
        o_ref[...] = (jnp.dot(h1, w2_[...], preferred_element_type=jnp.float32)
                      + b2_[...] + h_ref[...])

    return pl.pallas_call(
        body,
        grid=(nb,),
        in_specs=[pl.BlockSpec((_BN, _D), lambda i: (i, 0)),
                  pl.BlockSpec((_BN, _D), lambda i: (i, 0)),
                  pl.BlockSpec((_BN, _D), lambda i: (i + nb, 0)),
                  pl.BlockSpec((_BN, _D), lambda i: (i, 0)),
                  pl.BlockSpec((_BN, _D), lambda i: (i + nb, 0)),
                  _const_spec((_D, _D)), _const_spec((_D, _D)),
                  _const_spec((1, _D)), _const_spec((1, _D)),
                  _const_spec((1, _D)), _const_spec((_D, _D)),
                  _const_spec((1, _D))],
        out_specs=pl.BlockSpec((_BN, _D), lambda i: (i, 0)),
        out_shape=jax.ShapeDtypeStruct((_N, _D), jnp.float32),
    )(h, pa, pa, pb, pb, wh, wa, b1, g1, t1, w2, b2)


def _edgeupd_pl(e, g, we, ws, wd, b1, g1, t1, w2, b2):
    nb = _EH // _BE

    def body(e_ref, ns_ref, nd_ref, we_, ws_, wd_, b1_, g1_, t1_, w2_, b2_,
             o_ref):
        h1 = (jnp.dot(e_ref[...], we_[...], preferred_element_type=jnp.float32)
              + jnp.dot(ns_ref[...], ws_[...], preferred_element_type=jnp.float32)
              + jnp.dot(nd_ref[...], wd_[...], preferred_element_type=jnp.float32)
              + b1_[...])
        h1 = _ln_silu(h1, g1_[...], t1_[...])
        o_ref[...] = (jnp.dot(h1, w2_[...], preferred_element_type=jnp.float32)
                      + b2_[...] + e_ref[...])

    return pl.pallas_call(
        body,
        grid=(nb,),
        in_specs=[pl.BlockSpec((_BE, _D), lambda i: (i, 0)),
                  pl.BlockSpec((_BE, _D), lambda i: (i, 0)),
                  pl.BlockSpec((_BE, _D), lambda i: (i + nb, 0)),
                  _const_spec((_D, _D)), _const_spec((_D, _D)),
                  _const_spec((_D, _D)), _const_spec((1, _D)),
                  _const_spec((1, _D)), _const_spec((1, _D)),
                  _const_spec((_D, _D)), _const_spec((1, _D))],
        out_specs=pl.BlockSpec((_BE, _D), lambda i: (i, 0)),
        out_shape=jax.ShapeDtypeStruct((_EH, _D), jnp.float32),
    )(e, g, g, we, ws, wd, b1, g1, t1, w2, b2)


def _sc_gather(h, idx):
    b = idx.shape[0]
    per_w = b // _NW
    nch = per_w // _GCH
    lead = nch % 2
    mesh = plsc.VectorSubcoreMesh(core_axis_name="c", subcore_axis_name="s")

    @functools.partial(
        pl.kernel, mesh=mesh,
        out_type=jax.ShapeDtypeStruct((b, _D), jnp.float32),
        scratch_types=[pltpu.VMEM((per_w,), jnp.int32),
                       pltpu.VMEM((_GCH, _D), jnp.float32),
                       pltpu.VMEM((_GCH, _D), jnp.float32),
                       pltpu.SemaphoreType.DMA, pltpu.SemaphoreType.DMA,
                       pltpu.SemaphoreType.DMA, pltpu.SemaphoreType.DMA],
    )
    def k(h_hbm, idx_hbm, out_hbm, idx_v, r0, r1, g0, g1, w0, w1):
        wid = lax.axis_index("s") * 2 + lax.axis_index("c")
        base = wid * per_w
        pltpu.sync_copy(idx_hbm.at[pl.ds(base, per_w)], idx_v)
        rows = (r0, r1)
        gsem = (g0, g1)
        wsem = (w0, w1)

        if lead:
            pltpu.async_copy(h_hbm.at[idx_v.at[pl.ds(0, _GCH)]], r0, g0).wait()
            pltpu.sync_copy(r0, out_hbm.at[pl.ds(base, _GCH)])

        def outer(j, carry):
            for t in range(2):
                i = lead + j * 2 + t

                @pl.when(j > 0)
                def _():
                    pltpu.make_async_copy(
                        rows[t], out_hbm.at[pl.ds(base + i * _GCH, _GCH)],
                        wsem[t]).wait()

                pltpu.async_copy(
                    h_hbm.at[idx_v.at[pl.ds(i * _GCH, _GCH)]], rows[t],
                    gsem[t])
            for t in range(2):
                i = lead + j * 2 + t
                pltpu.make_async_copy(
                    h_hbm.at[idx_v.at[pl.ds(i * _GCH, _GCH)]], rows[t],
                    gsem[t]).wait()
                pltpu.async_copy(rows[t],
                                 out_hbm.at[pl.ds(base + i * _GCH, _GCH)],
                                 wsem[t])
            return carry

        lax.fori_loop(0, (nch - lead) // 2, outer, 0)
        for t in range(2):
            pltpu.make_async_copy(
                rows[t], out_hbm.at[pl.ds(base, _GCH)], wsem[t]).wait()

    return k(h, idx)


def _sc_scatter(m, dsth, zer):
    per_w = _EH // _NW
    nch = per_w // _SCH
    stripe = 624
    tail = _N - 16 * stripe
    mesh = plsc.VectorSubcoreMesh(core_axis_name="c", subcore_axis_name="s")

    @functools.partial(
        pl.kernel, mesh=mesh,
        out_type=jax.ShapeDtypeStruct((2 * _N, _D), jnp.float32),
        scratch_types=[pltpu.VMEM((_SCH,), jnp.int32),
                       pltpu.VMEM((_SCH,), jnp.int32),
                       pltpu.VMEM((_SCH, _D), jnp.float32),
                       pltpu.VMEM((_SCH, _D), jnp.float32),
                       pltpu.VMEM_SHARED((_N, _D), jnp.float32),
                       pltpu.SemaphoreType.DMA, pltpu.SemaphoreType.DMA,
                       pltpu.SemaphoreType.DMA, pltpu.SemaphoreType.DMA],
    )
    def k(m_hbm, dst_hbm, zer_hbm, out_hbm, i0, i1, r0, r1, acc,
          mi0, mi1, mr0, mr1):
        cid = lax.axis_index("c")
        sid = lax.axis_index("s")
        wid = sid * 2 + cid
        pltpu.sync_copy(zer_hbm.at[pl.ds(sid * stripe, stripe)],
                        acc.at[pl.ds(sid * stripe, stripe)])

        @pl.when(sid == 15)
        def _():
            pltpu.sync_copy(zer_hbm.at[pl.ds(16 * stripe, tail)],
                            acc.at[pl.ds(16 * stripe, tail)])

        plsc.subcore_barrier()

        base = wid * per_w
        idx_b = (i0, i1)
        rows_b = (r0, r1)
        isem = (mi0, mi1)
        msem = (mr0, mr1)

        lead = nch % 2
        if lead:
            pltpu.sync_copy(dst_hbm.at[pl.ds(base, _SCH)], i0)
            pltpu.sync_copy(m_hbm.at[pl.ds(base, _SCH)], r0)
            pltpu.sync_copy(r0, acc.at[i0], add=True)

        def outer(j, carry):
            for t in range(2):
                i = lead + j * 2 + t
                off = base + i * _SCH
                pltpu.async_copy(dst_hbm.at[pl.ds(off, _SCH)], idx_b[t],
                                 isem[t])
                pltpu.async_copy(m_hbm.at[pl.ds(off, _SCH)], rows_b[t],
                                 msem[t])
            for t in range(2):
                i = lead + j * 2 + t
                off = base + i * _SCH
                pltpu.make_async_copy(dst_hbm.at[pl.ds(off, _SCH)], idx_b[t],
                                      isem[t]).wait()
                pltpu.make_async_copy(m_hbm.at[pl.ds(off, _SCH)], rows_b[t],
                                      msem[t]).wait()
                pltpu.sync_copy(rows_b[t], acc.at[idx_b[t]], add=True)
            return carry

        lax.fori_loop(0, (nch - lead) // 2, outer, 0)
        plsc.subcore_barrier()
        pltpu.sync_copy(
            acc.at[pl.ds(sid * stripe, stripe)],
            out_hbm.at[pl.ds(cid * _N + sid * stripe, stripe)])

        @pl.when(sid == 15)
        def _():
            pltpu.sync_copy(
                acc.at[pl.ds(16 * stripe, tail)],
                out_hbm.at[pl.ds(cid * _N + 16 * stripe, tail)])

    return k(m, dsth, zer)


def _pack_layer(lp):
    mp, gp, np_, ep = lp["message"], lp["gate"], lp["node"], lp["edge"]
    w1 = jnp.concatenate([mp["l1"]["W"], gp["l1"]["W"]], axis=1)
    msg = dict(
        wd=w1[:_D], ws=w1[_D:2 * _D], we=w1[2 * _D:],
        b1=jnp.concatenate([mp["l1"]["b"], gp["l1"]["b"]]).reshape(1, 2 * _D),
        g1=jnp.concatenate([mp["ln_g"], gp["ln_g"]]).reshape(1, 2 * _D),
        t1=jnp.concatenate([mp["ln_b"], gp["ln_b"]]).reshape(1, 2 * _D),
        w2m=mp["l2"]["W"], b2m=mp["l2"]["b"].reshape(1, _D),
        w2g=jnp.broadcast_to(gp["l2"]["W"], (_D, _D)),
        b2g=jnp.broadcast_to(gp["l2"]["b"].reshape(1, 1), (1, _D)),
    )
    node = dict(
        wh=np_["l1"]["W"][:_D], wa=np_["l1"]["W"][_D:],
        b1=np_["l1"]["b"].reshape(1, _D),
        g1=np_["ln_g"].reshape(1, _D), t1=np_["ln_b"].reshape(1, _D),
        w2=np_["l2"]["W"], b2=np_["l2"]["b"].reshape(1, _D),
    )
    edge = dict(
        we=ep["l1"]["W"][:_D], ws=ep["l1"]["W"][_D:2 * _D],
        wd=ep["l1"]["W"][2 * _D:],
        b1=ep["l1"]["b"].reshape(1, _D),
        g1=ep["ln_g"].reshape(1, _D), t1=ep["ln_b"].reshape(1, _D),
        w2=ep["l2"]["W"], b2=ep["l2"]["b"].reshape(1, _D),
    )
    return msg, node, edge


def kernel(x, edge_index, edge_attr, params):
    src = edge_index[0]
    dst = edge_index[1]
    dst_a, dst_b = dst[:_EH], dst[_EH:]
    idx_a = jnp.concatenate([src[:_EH], dst_a])
    idx_b = jnp.concatenate([src[_EH:], dst_b])
    zer = jnp.zeros((_N, _D), jnp.float32)

    h = _linear_pl(x, params["node_enc"]["W"], params["node_enc"]["b"], _BN)
    e_a = _linear_pl(edge_attr[:_EH], params["edge_enc"]["W"],
                     params["edge_enc"]["b"], _BE)
    e_b = _linear_pl(edge_attr[_EH:], params["edge_enc"]["W"],
                     params["edge_enc"]["b"], _BE)

    g_a = _sc_gather(h, idx_a)
    g_b = _sc_gather(h, idx_b)
    for lp in params["layers"]:
        msg, node, edge = _pack_layer(lp)
        m_a = _message_pl(g_a, e_a, **msg)
        parts_a = _sc_scatter(m_a, dst_a, zer)
        m_b = _message_pl(g_b, e_b, **msg)
        parts_b = _sc_scatter(m_b, dst_b, zer)
        h = _node_pl(h, parts_a, parts_b, **node)
        g_a = _sc_gather(h, idx_a)
        e_a = _edgeupd_pl(e_a, g_a, **edge)
        g_b = _sc_gather(h, idx_b)
        e_b = _edgeupd_pl(e_b, g_b, **edge)

    x_out = _linear_pl(h, params["node_dec"]["W"], params["node_dec"]["b"], _BN)
    e_out_a = _linear_pl(e_a, params["edge_dec"]["W"],
                         params["edge_dec"]["b"], _BE)
    e_out_b = _linear_pl(e_b, params["edge_dec"]["W"],
                         params["edge_dec"]["b"], _BE)
    return (x_out, jnp.concatenate([e_out_a, e_out_b], axis=0))


# speedup vs baseline: 1.3577x; 1.0228x over previous
import functools

import jax
import jax.numpy as jnp
from jax import lax
from jax.experimental import pallas as pl
from jax.experimental.pallas import tpu as pltpu
from jax.experimental.pallas import tpu_sc as plsc

_N = 10000
_E = 320000
_EH = _E // 2
_D = 128
_LN_EPS = 1e-5

_BE = 8000
_BN = 2000
_NW = 32
_GCH = 80
_SCH = 40


def _sig(x):
    return 0.5 * jnp.tanh(0.5 * x) + 0.5


def _ln_silu(h, g, b):
    mu = jnp.mean(h, axis=-1, keepdims=True)
    c = h - mu
    var = jnp.mean(c * c, axis=-1, keepdims=True)
    hn = c * lax.rsqrt(var + _LN_EPS) * g + b
    return hn * _sig(hn)


def _const_spec(shape):
    return pl.BlockSpec(shape, lambda i: tuple(0 for _ in shape))


def _linear_pl(x, w, b, block):
    n, k = x.shape
    m = w.shape[1]

    def body(x_ref, w_ref, b_ref, o_ref):
        o_ref[...] = (
            jnp.dot(x_ref[...], w_ref[...], preferred_element_type=jnp.float32)
            + b_ref[...])

    return pl.pallas_call(
        body,
        grid=(n // block,),
        in_specs=[pl.BlockSpec((block, k), lambda i: (i, 0)),
                  _const_spec((k, m)),
                  _const_spec((1, m))],
        out_specs=pl.BlockSpec((block, m), lambda i: (i, 0)),
        out_shape=jax.ShapeDtypeStruct((n, m), jnp.float32),
    )(x, w, b.reshape(1, m))


def _message_pl(g, e, wd, ws, we, b1, g1, t1, w2m, b2m, w2g, b2g):
    nb = _EH // _BE

    def body(hd_ref, hs_ref, e_ref, wd_, ws_, we_, b1_, g1_, t1_,
             w2m_, b2m_, w2g_, b2g_, o_ref):
        h1 = (jnp.dot(hd_ref[...], wd_[...], preferred_element_type=jnp.float32)
              + jnp.dot(hs_ref[...], ws_[...], preferred_element_type=jnp.float32)
              + jnp.dot(e_ref[...], we_[...], preferred_element_type=jnp.float32)
              + b1_[...])
        hm = _ln_silu(h1[:, :_D], g1_[:, :_D], t1_[:, :_D])
        hg = _ln_silu(h1[:, _D:], g1_[:, _D:], t1_[:, _D:])
        msg = jnp.dot(hm, w2m_[...], preferred_element_type=jnp.float32) + b2m_[...]
        gl = jnp.dot(hg, w2g_[...], preferred_element_type=jnp.float32) + b2g_[...]
        o_ref[...] = _sig(gl) * msg

    return pl.pallas_call(
        body,
        grid=(nb,),
        in_specs=[pl.BlockSpec((_BE, _D), lambda i: (i + nb, 0)),
                  pl.BlockSpec((_BE, _D), lambda i: (i, 0)),
                  pl.BlockSpec((_BE, _D), lambda i: (i, 0)),
                  _const_spec((_D, 2 * _D)), _const_spec((_D, 2 * _D)),
                  _const_spec((_D, 2 * _D)), _const_spec((1, 2 * _D)),
                  _const_spec((1, 2 * _D)), _const_spec((1, 2 * _D)),
                  _const_spec((_D, _D)), _const_spec((1, _D)),
                  _const_spec((_D, _D)), _const_spec((1, _D))],
        out_specs=pl.BlockSpec((_BE, _D), lambda i: (i, 0)),
        out_shape=jax.ShapeDtypeStruct((_EH, _D), jnp.float32),
    )(g, g, e, wd, ws, we, b1, g1, t1, w2m, b2m, w2g, b2g)


def _node_pl(h, pa, pb, wh, wa, b1, g1, t1, w2, b2):
    nb = _N // _BN

    def body(h_ref, pa0_ref, pa1_ref, pb0_ref, pb1_ref,
             wh_, wa_, b1_, g1_, t1_, w2_, b2_, o_ref):
        aggr = (pa0_ref[...] + pa1_ref[...]) + (pb0_ref[...] + pb1_ref[...])
        h1 = (jnp.dot(h_ref[...], wh_[...], preferred_element_type=jnp.float32)
              + jnp.dot(aggr, wa_[...], preferred_element_type=jnp.float32)
              + b1_[...])
        h1 = _ln_silu(h1, g1_[...], t1_[...])
        o_ref[...] = (jnp.dot(h1, w2_[...], preferred_element_type=jnp.float32)
                      + b2_[...] + h_ref[...])

    return pl.pallas_call(
        body,
        grid=(nb,),
        in_specs=[pl.BlockSpec((_BN, _D), lambda i: (i, 0)),
                  pl.BlockSpec((_BN, _D), lambda i: (i, 0)),
                  pl.BlockSpec((_BN, _D), lambda i: (i + nb, 0)),
                  pl.BlockSpec((_BN, _D), lambda i: (i, 0)),
                  pl.BlockSpec((_BN, _D), lambda i: (i + nb, 0)),
                  _const_spec((_D, _D)), _const_spec((_D, _D)),
                  _const_spec((1, _D)), _const_spec((1, _D)),
                  _const_spec((1, _D)), _const_spec((_D, _D)),
                  _const_spec((1, _D))],
        out_specs=pl.BlockSpec((_BN, _D), lambda i: (i, 0)),
        out_shape=jax.ShapeDtypeStruct((_N, _D), jnp.float32),
    )(h, pa, pa, pb, pb, wh, wa, b1, g1, t1, w2, b2)


def _edgeupd_pl(e, g, we, ws, wd, b1, g1, t1, w2, b2):
    nb = _EH // _BE

    def body(e_ref, ns_ref, nd_ref, we_, ws_, wd_, b1_, g1_, t1_, w2_, b2_,
             o_ref):
        h1 = (jnp.dot(e_ref[...], we_[...], preferred_element_type=jnp.float32)
              + jnp.dot(ns_ref[...], ws_[...], preferred_element_type=jnp.float32)
              + jnp.dot(nd_ref[...], wd_[...], preferred_element_type=jnp.float32)
              + b1_[...])
        h1 = _ln_silu(h1, g1_[...], t1_[...])
        o_ref[...] = (jnp.dot(h1, w2_[...], preferred_element_type=jnp.float32)
                      + b2_[...] + e_ref[...])

    return pl.pallas_call(
        body,
        grid=(nb,),
        in_specs=[pl.BlockSpec((_BE, _D), lambda i: (i, 0)),
                  pl.BlockSpec((_BE, _D), lambda i: (i, 0)),
                  pl.BlockSpec((_BE, _D), lambda i: (i + nb, 0)),
                  _const_spec((_D, _D)), _const_spec((_D, _D)),
                  _const_spec((_D, _D)), _const_spec((1, _D)),
                  _const_spec((1, _D)), _const_spec((1, _D)),
                  _const_spec((_D, _D)), _const_spec((1, _D))],
        out_specs=pl.BlockSpec((_BE, _D), lambda i: (i, 0)),
        out_shape=jax.ShapeDtypeStruct((_EH, _D), jnp.float32),
    )(e, g, g, we, ws, wd, b1, g1, t1, w2, b2)


def _sc_gather(h, idx):
    b = idx.shape[0]
    per_w = b // _NW
    nch = per_w // _GCH
    lead = nch % 2
    mesh = plsc.VectorSubcoreMesh(core_axis_name="c", subcore_axis_name="s")

    @functools.partial(
        pl.kernel, mesh=mesh,
        out_type=jax.ShapeDtypeStruct((b, _D), jnp.float32),
        scratch_types=[pltpu.VMEM((per_w,), jnp.int32),
                       pltpu.VMEM((_GCH, _D), jnp.float32),
                       pltpu.VMEM((_GCH, _D), jnp.float32),
                       pltpu.SemaphoreType.DMA, pltpu.SemaphoreType.DMA,
                       pltpu.SemaphoreType.DMA, pltpu.SemaphoreType.DMA],
    )
    def k(h_hbm, idx_hbm, out_hbm, idx_v, r0, r1, g0, g1, w0, w1):
        wid = lax.axis_index("s") * 2 + lax.axis_index("c")
        base = wid * per_w
        pltpu.sync_copy(idx_hbm.at[pl.ds(base, per_w)], idx_v)
        rows = (r0, r1)
        gsem = (g0, g1)
        wsem = (w0, w1)

        if lead:
            pltpu.async_copy(h_hbm.at[idx_v.at[pl.ds(0, _GCH)]], r0, g0).wait()
            pltpu.sync_copy(r0, out_hbm.at[pl.ds(base, _GCH)])

        def outer(j, carry):
            for t in range(2):
                i = lead + j * 2 + t

                @pl.when(j > 0)
                def _():
                    pltpu.make_async_copy(
                        rows[t], out_hbm.at[pl.ds(base + i * _GCH, _GCH)],
                        wsem[t]).wait()

                pltpu.async_copy(
                    h_hbm.at[idx_v.at[pl.ds(i * _GCH, _GCH)]], rows[t],
                    gsem[t])
            for t in range(2):
                i = lead + j * 2 + t
                pltpu.make_async_copy(
                    h_hbm.at[idx_v.at[pl.ds(i * _GCH, _GCH)]], rows[t],
                    gsem[t]).wait()
                pltpu.async_copy(rows[t],
                                 out_hbm.at[pl.ds(base + i * _GCH, _GCH)],
                                 wsem[t])
            return carry

        lax.fori_loop(0, (nch - lead) // 2, outer, 0)
        for t in range(2):
            pltpu.make_async_copy(
                rows[t], out_hbm.at[pl.ds(base, _GCH)], wsem[t]).wait()

    return k(h, idx)


def _sc_scatter(m, dsth, zer):
    per_w = _EH // _NW
    nch = per_w // _SCH
    stripe = 624
    tail = _N - 16 * stripe
    mesh = plsc.VectorSubcoreMesh(core_axis_name="c", subcore_axis_name="s")

    @functools.partial(
        pl.kernel, mesh=mesh,
        out_type=jax.ShapeDtypeStruct((2 * _N, _D), jnp.float32),
        scratch_types=[pltpu.VMEM((_SCH,), jnp.int32),
                       pltpu.VMEM((_SCH,), jnp.int32),
                       pltpu.VMEM((_SCH, _D), jnp.float32),
                       pltpu.VMEM((_SCH, _D), jnp.float32),
                       pltpu.VMEM_SHARED((_N, _D), jnp.float32),
                       pltpu.SemaphoreType.DMA, pltpu.SemaphoreType.DMA,
                       pltpu.SemaphoreType.DMA, pltpu.SemaphoreType.DMA],
    )
    def k(m_hbm, dst_hbm, zer_hbm, out_hbm, i0, i1, r0, r1, acc,
          mi0, mi1, mr0, mr1):
        cid = lax.axis_index("c")
        sid = lax.axis_index("s")
        wid = sid * 2 + cid
        pltpu.sync_copy(zer_hbm.at[pl.ds(sid * stripe, stripe)],
                        acc.at[pl.ds(sid * stripe, stripe)])

        @pl.when(sid == 15)
        def _():
            pltpu.sync_copy(zer_hbm.at[pl.ds(16 * stripe, tail)],
                            acc.at[pl.ds(16 * stripe, tail)])

        plsc.subcore_barrier()

        base = wid * per_w
        idx_b = (i0, i1)
        rows_b = (r0, r1)
        isem = (mi0, mi1)
        msem = (mr0, mr1)

        lead = nch % 2
        if lead:
            pltpu.sync_copy(dst_hbm.at[pl.ds(base, _SCH)], i0)
            pltpu.sync_copy(m_hbm.at[pl.ds(base, _SCH)], r0)
            pltpu.sync_copy(r0, acc.at[i0], add=True)

        def outer(j, carry):
            for t in range(2):
                i = lead + j * 2 + t
                off = base + i * _SCH
                pltpu.async_copy(dst_hbm.at[pl.ds(off, _SCH)], idx_b[t],
                                 isem[t])
                pltpu.async_copy(m_hbm.at[pl.ds(off, _SCH)], rows_b[t],
                                 msem[t])
            for t in range(2):
                i = lead + j * 2 + t
                off = base + i * _SCH
                pltpu.make_async_copy(dst_hbm.at[pl.ds(off, _SCH)], idx_b[t],
                                      isem[t]).wait()
                pltpu.make_async_copy(m_hbm.at[pl.ds(off, _SCH)], rows_b[t],
                                      msem[t]).wait()
                pltpu.sync_copy(rows_b[t], acc.at[idx_b[t]], add=True)
            return carry

        lax.fori_loop(0, (nch - lead) // 2, outer, 0)
        plsc.subcore_barrier()
        pltpu.sync_copy(
            acc.at[pl.ds(sid * stripe, stripe)],
            out_hbm.at[pl.ds(cid * _N + sid * stripe, stripe)])

        @pl.when(sid == 15)
        def _():
            pltpu.sync_copy(
                acc.at[pl.ds(16 * stripe, tail)],
                out_hbm.at[pl.ds(cid * _N + 16 * stripe, tail)])

    return k(m, dsth, zer)


def _pack_layer(lp):
    mp, gp, np_, ep = lp["message"], lp["gate"], lp["node"], lp["edge"]
    w1 = jnp.concatenate([mp["l1"]["W"], gp["l1"]["W"]], axis=1)
    msg = dict(
        wd=w1[:_D], ws=w1[_D:2 * _D], we=w1[2 * _D:],
        b1=jnp.concatenate([mp["l1"]["b"], gp["l1"]["b"]]).reshape(1, 2 * _D),
        g1=jnp.concatenate([mp["ln_g"], gp["ln_g"]]).reshape(1, 2 * _D),
        t1=jnp.concatenate([mp["ln_b"], gp["ln_b"]]).reshape(1, 2 * _D),
        w2m=mp["l2"]["W"], b2m=mp["l2"]["b"].reshape(1, _D),
        w2g=jnp.broadcast_to(gp["l2"]["W"], (_D, _D)),
        b2g=jnp.broadcast_to(gp["l2"]["b"].reshape(1, 1), (1, _D)),
    )
    node = dict(
        wh=np_["l1"]["W"][:_D], wa=np_["l1"]["W"][_D:],
        b1=np_["l1"]["b"].reshape(1, _D),
        g1=np_["ln_g"].reshape(1, _D), t1=np_["ln_b"].reshape(1, _D),
        w2=np_["l2"]["W"], b2=np_["l2"]["b"].reshape(1, _D),
    )
    edge = dict(
        we=ep["l1"]["W"][:_D], ws=ep["l1"]["W"][_D:2 * _D],
        wd=ep["l1"]["W"][2 * _D:],
        b1=ep["l1"]["b"].reshape(1, _D),
        g1=ep["ln_g"].reshape(1, _D), t1=ep["ln_b"].reshape(1, _D),
        w2=ep["l2"]["W"], b2=ep["l2"]["b"].reshape(1, _D),
    )
    return msg, node, edge


def kernel(x, edge_index, edge_attr, params):
    src = edge_index[0]
    dst = edge_index[1]
    dst_a, dst_b = dst[:_EH], dst[_EH:]
    idx_a = jnp.concatenate([src[:_EH], dst_a])
    idx_b = jnp.concatenate([src[_EH:], dst_b])
    zer = jnp.zeros((_N, _D), jnp.float32)

    h = _linear_pl(x, params["node_enc"]["W"], params["node_enc"]["b"], _BN)
    e_a = _linear_pl(edge_attr[:_EH], params["edge_enc"]["W"],
                     params["edge_enc"]["b"], _BE)
    e_b = _linear_pl(edge_attr[_EH:], params["edge_enc"]["W"],
                     params["edge_enc"]["b"], _BE)

    g_a = _sc_gather(h, idx_a)
    g_b = _sc_gather(h, idx_b)
    for lp in params["layers"]:
        msg, node, edge = _pack_layer(lp)
        m_a = _message_pl(g_a, e_a, **msg)
        parts_a = _sc_scatter(m_a, dst_a, zer)
        m_b = _message_pl(g_b, e_b, **msg)
        parts_b = _sc_scatter(m_b, dst_b, zer)
        h = _node_pl(h, parts_a, parts_b, **node)
        g_a = _sc_gather(h, idx_a)
        e_a = _edgeupd_pl(e_a, g_a, **edge)
        g_b = _sc_gather(h, idx_b)
        e_b = _edgeupd_pl(e_b, g_b, **edge)

    x_out = _linear_pl(h, params["node_dec"]["W"], params["node_dec"]["b"], _BN)
    e_out_a = _linear_pl(e_a, params["edge_dec"]["W"],
                         params["edge_dec"]["b"], _BE)
    e_out_b = _linear_pl(e_b, params["edge_dec"]["W"],
                         params["edge_dec"]["b"], _BE)
    return (x_out, jnp.concatenate([e_out_a, e_out_b], axis=0))


# speedup vs baseline: 1.3829x; 1.0185x over previous
import functools

import jax
import jax.numpy as jnp
from jax import lax
from jax.experimental import pallas as pl
from jax.experimental.pallas import tpu as pltpu
from jax.experimental.pallas import tpu_sc as plsc

_N = 10000
_E = 320000
_EH = _E // 2
_D = 128
_LN_EPS = 1e-5

_BE = 8000
_BN = 2000
_NW = 32
_GCH = 80
_SCH = 40


def _sig(x):
    return 0.5 * jnp.tanh(0.5 * x) + 0.5


def _ln_silu(h, g, b):
    mu = jnp.mean(h, axis=-1, keepdims=True)
    c = h - mu
    var = jnp.mean(c * c, axis=-1, keepdims=True)
    hn = c * lax.rsqrt(var + _LN_EPS) * g + b
    return hn * _sig(hn)


def _const_spec(shape):
    return pl.BlockSpec(shape, lambda i: tuple(0 for _ in shape))


def _linear_pl(x, w, b, block):
    n, k = x.shape
    m = w.shape[1]

    def body(x_ref, w_ref, b_ref, o_ref):
        o_ref[...] = (
            jnp.dot(x_ref[...], w_ref[...], preferred_element_type=jnp.float32)
            + b_ref[...])

    return pl.pallas_call(
        body,
        grid=(n // block,),
        in_specs=[pl.BlockSpec((block, k), lambda i: (i, 0)),
                  _const_spec((k, m)),
                  _const_spec((1, m))],
        out_specs=pl.BlockSpec((block, m), lambda i: (i, 0)),
        out_shape=jax.ShapeDtypeStruct((n, m), jnp.float32),
    )(x, w, b.reshape(1, m))


def _message_pl(g, e, wd, ws, we, b1, g1, t1, w2m, b2m, w2g, b2g):
    nb = _EH // _BE

    def body(hd_ref, hs_ref, e_ref, wd_, ws_, we_, b1_, g1_, t1_,
             w2m_, b2m_, w2g_, b2g_, o_ref):
        h1 = (jnp.dot(hd_ref[...], wd_[...], preferred_element_type=jnp.float32)
              + jnp.dot(hs_ref[...], ws_[...], preferred_element_type=jnp.float32)
              + jnp.dot(e_ref[...], we_[...], preferred_element_type=jnp.float32)
              + b1_[...])
        hm = _ln_silu(h1[:, :_D], g1_[:, :_D], t1_[:, :_D])
        hg = _ln_silu(h1[:, _D:], g1_[:, _D:], t1_[:, _D:])
        msg = jnp.dot(hm, w2m_[...], preferred_element_type=jnp.float32) + b2m_[...]
        gl = jnp.dot(hg, w2g_[...], preferred_element_type=jnp.float32) + b2g_[...]
        o_ref[...] = _sig(gl) * msg

    return pl.pallas_call(
        body,
        grid=(nb,),
        in_specs=[pl.BlockSpec((_BE, _D), lambda i: (i + nb, 0)),
                  pl.BlockSpec((_BE, _D), lambda i: (i, 0)),
                  pl.BlockSpec((_BE, _D), lambda i: (i, 0)),
                  _const_spec((_D, 2 * _D)), _const_spec((_D, 2 * _D)),
                  _const_spec((_D, 2 * _D)), _const_spec((1, 2 * _D)),
                  _const_spec((1, 2 * _D)), _const_spec((1, 2 * _D)),
                  _const_spec((_D, _D)), _const_spec((1, _D)),
                  _const_spec((_D, _D)), _const_spec((1, _D))],
        out_specs=pl.BlockSpec((_BE, _D), lambda i: (i, 0)),
        out_shape=jax.ShapeDtypeStruct((_EH, _D), jnp.float32),
    )(g, g, e, wd, ws, we, b1, g1, t1, w2m, b2m, w2g, b2g)


def _node_pl(h, pa, pb, wh, wa, b1, g1, t1, w2, b2):
    nb = _N // _BN

    def body(h_ref, pa0_ref, pa1_ref, pb0_ref, pb1_ref,
             wh_, wa_, b1_, g1_, t1_, w2_, b2_, o_ref):
        aggr = (pa0_ref[...] + pa1_ref[...]) + (pb0_ref[...] + pb1_ref[...])
        h1 = (jnp.dot(h_ref[...], wh_[...], preferred_element_type=jnp.float32)
              + jnp.dot(aggr, wa_[...], preferred_element_type=jnp.float32)
              + b1_[...])
        h1 = _ln_silu(h1, g1_[...], t1_[...])
        o_ref[...] = (jnp.dot(h1, w2_[...], preferred_element_type=jnp.float32)
                      + b2_[...] + h_ref[...])

    return pl.pallas_call(
        body,
        grid=(nb,),
        in_specs=[pl.BlockSpec((_BN, _D), lambda i: (i, 0)),
                  pl.BlockSpec((_BN, _D), lambda i: (i, 0)),
                  pl.BlockSpec((_BN, _D), lambda i: (i + nb, 0)),
                  pl.BlockSpec((_BN, _D), lambda i: (i, 0)),
                  pl.BlockSpec((_BN, _D), lambda i: (i + nb, 0)),
                  _const_spec((_D, _D)), _const_spec((_D, _D)),
                  _const_spec((1, _D)), _const_spec((1, _D)),
                  _const_spec((1, _D)), _const_spec((_D, _D)),
                  _const_spec((1, _D))],
        out_specs=pl.BlockSpec((_BN, _D), lambda i: (i, 0)),
        out_shape=jax.ShapeDtypeStruct((_N, _D), jnp.float32),
    )(h, pa, pa, pb, pb, wh, wa, b1, g1, t1, w2, b2)


def _edgeupd_pl(e, g, we, ws, wd, b1, g1, t1, w2, b2):
    nb = _EH // _BE

    def body(e_ref, ns_ref, nd_ref, we_, ws_, wd_, b1_, g1_, t1_, w2_, b2_,
             o_ref):
        h1 = (jnp.dot(e_ref[...], we_[...], preferred_element_type=jnp.float32)
              + jnp.dot(ns_ref[...], ws_[...], preferred_element_type=jnp.float32)
              + jnp.dot(nd_ref[...], wd_[...], preferred_element_type=jnp.float32)
              + b1_[...])
        h1 = _ln_silu(h1, g1_[...], t1_[...])
        o_ref[...] = (jnp.dot(h1, w2_[...], preferred_element_type=jnp.float32)
                      + b2_[...] + e_ref[...])

    return pl.pallas_call(
        body,
        grid=(nb,),
        in_specs=[pl.BlockSpec((_BE, _D), lambda i: (i, 0)),
                  pl.BlockSpec((_BE, _D), lambda i: (i, 0)),
                  pl.BlockSpec((_BE, _D), lambda i: (i + nb, 0)),
                  _const_spec((_D, _D)), _const_spec((_D, _D)),
                  _const_spec((_D, _D)), _const_spec((1, _D)),
                  _const_spec((1, _D)), _const_spec((1, _D)),
                  _const_spec((_D, _D)), _const_spec((1, _D))],
        out_specs=pl.BlockSpec((_BE, _D), lambda i: (i, 0)),
        out_shape=jax.ShapeDtypeStruct((_EH, _D), jnp.float32),
    )(e, g, g, we, ws, wd, b1, g1, t1, w2, b2)


def _sc_gather(h, idx):
    b = idx.shape[0]
    per_w = b // _NW
    nch = per_w // _GCH
    nbuf = 4
    lead = nch % nbuf
    mesh = plsc.VectorSubcoreMesh(core_axis_name="c", subcore_axis_name="s")

    @functools.partial(
        pl.kernel, mesh=mesh,
        out_type=jax.ShapeDtypeStruct((b, _D), jnp.float32),
        scratch_types=[pltpu.VMEM((per_w,), jnp.int32)]
        + [pltpu.VMEM((_GCH, _D), jnp.float32)] * nbuf
        + [pltpu.SemaphoreType.DMA] * (2 * nbuf),
    )
    def k(h_hbm, idx_hbm, out_hbm, idx_v, *bufs):
        rows = bufs[:nbuf]
        gsem = bufs[nbuf:2 * nbuf]
        wsem = bufs[2 * nbuf:]
        wid = lax.axis_index("s") * 2 + lax.axis_index("c")
        base = wid * per_w
        pltpu.sync_copy(idx_hbm.at[pl.ds(base, per_w)], idx_v)

        for t in range(lead):
            pltpu.async_copy(h_hbm.at[idx_v.at[pl.ds(t * _GCH, _GCH)]],
                             rows[t], gsem[t]).wait()
            pltpu.sync_copy(rows[t],
                            out_hbm.at[pl.ds(base + t * _GCH, _GCH)])

        def outer(j, carry):
            for t in range(nbuf):
                i = lead + j * nbuf + t

                @pl.when(j > 0)
                def _():
                    pltpu.make_async_copy(
                        rows[t], out_hbm.at[pl.ds(base + i * _GCH, _GCH)],
                        wsem[t]).wait()

                pltpu.async_copy(
                    h_hbm.at[idx_v.at[pl.ds(i * _GCH, _GCH)]], rows[t],
                    gsem[t])
            for t in range(nbuf):
                i = lead + j * nbuf + t
                pltpu.make_async_copy(
                    h_hbm.at[idx_v.at[pl.ds(i * _GCH, _GCH)]], rows[t],
                    gsem[t]).wait()
                pltpu.async_copy(rows[t],
                                 out_hbm.at[pl.ds(base + i * _GCH, _GCH)],
                                 wsem[t])
            return carry

        lax.fori_loop(0, (nch - lead) // nbuf, outer, 0)
        for t in range(nbuf):
            pltpu.make_async_copy(
                rows[t], out_hbm.at[pl.ds(base, _GCH)], wsem[t]).wait()

    return k(h, idx)


def _sc_scatter(m, dsth, zer):
    per_w = _EH // _NW
    nch = per_w // _SCH
    stripe = 624
    tail = _N - 16 * stripe
    mesh = plsc.VectorSubcoreMesh(core_axis_name="c", subcore_axis_name="s")

    nbuf = 4

    @functools.partial(
        pl.kernel, mesh=mesh,
        out_type=jax.ShapeDtypeStruct((2 * _N, _D), jnp.float32),
        scratch_types=[pltpu.VMEM_SHARED((_N, _D), jnp.float32)]
        + [pltpu.VMEM((_SCH,), jnp.int32)] * nbuf
        + [pltpu.VMEM((_SCH, _D), jnp.float32)] * nbuf
        + [pltpu.SemaphoreType.DMA] * (2 * nbuf),
    )
    def k(m_hbm, dst_hbm, zer_hbm, out_hbm, acc, *bufs):
        idx_b = bufs[:nbuf]
        rows_b = bufs[nbuf:2 * nbuf]
        isem = bufs[2 * nbuf:3 * nbuf]
        msem = bufs[3 * nbuf:]
        cid = lax.axis_index("c")
        sid = lax.axis_index("s")
        wid = sid * 2 + cid
        pltpu.sync_copy(zer_hbm.at[pl.ds(sid * stripe, stripe)],
                        acc.at[pl.ds(sid * stripe, stripe)])

        @pl.when(sid == 15)
        def _():
            pltpu.sync_copy(zer_hbm.at[pl.ds(16 * stripe, tail)],
                            acc.at[pl.ds(16 * stripe, tail)])

        plsc.subcore_barrier()

        base = wid * per_w

        lead = nch % nbuf
        for t in range(lead):
            off = base + t * _SCH
            pltpu.sync_copy(dst_hbm.at[pl.ds(off, _SCH)], idx_b[t])
            pltpu.sync_copy(m_hbm.at[pl.ds(off, _SCH)], rows_b[t])
            pltpu.sync_copy(rows_b[t], acc.at[idx_b[t]], add=True)

        def outer(j, carry):
            for t in range(nbuf):
                i = lead + j * nbuf + t
                off = base + i * _SCH
                pltpu.async_copy(dst_hbm.at[pl.ds(off, _SCH)], idx_b[t],
                                 isem[t])
                pltpu.async_copy(m_hbm.at[pl.ds(off, _SCH)], rows_b[t],
                                 msem[t])
            for t in range(nbuf):
                i = lead + j * nbuf + t
                off = base + i * _SCH
                pltpu.make_async_copy(dst_hbm.at[pl.ds(off, _SCH)], idx_b[t],
                                      isem[t]).wait()
                pltpu.make_async_copy(m_hbm.at[pl.ds(off, _SCH)], rows_b[t],
                                      msem[t]).wait()
                pltpu.sync_copy(rows_b[t], acc.at[idx_b[t]], add=True)
            return carry

        lax.fori_loop(0, (nch - lead) // nbuf, outer, 0)
        plsc.subcore_barrier()
        pltpu.sync_copy(
            acc.at[pl.ds(sid * stripe, stripe)],
            out_hbm.at[pl.ds(cid * _N + sid * stripe, stripe)])

        @pl.when(sid == 15)
        def _():
            pltpu.sync_copy(
                acc.at[pl.ds(16 * stripe, tail)],
                out_hbm.at[pl.ds(cid * _N + 16 * stripe, tail)])

    return k(m, dsth, zer)


def _pack_layer(lp):
    mp, gp, np_, ep = lp["message"], lp["gate"], lp["node"], lp["edge"]
    w1 = jnp.concatenate([mp["l1"]["W"], gp["l1"]["W"]], axis=1)
    msg = dict(
        wd=w1[:_D], ws=w1[_D:2 * _D], we=w1[2 * _D:],
        b1=jnp.concatenate([mp["l1"]["b"], gp["l1"]["b"]]).reshape(1, 2 * _D),
        g1=jnp.concatenate([mp["ln_g"], gp["ln_g"]]).reshape(1, 2 * _D),
        t1=jnp.concatenate([mp["ln_b"], gp["ln_b"]]).reshape(1, 2 * _D),
        w2m=mp["l2"]["W"], b2m=mp["l2"]["b"].reshape(1, _D),
        w2g=jnp.broadcast_to(gp["l2"]["W"], (_D, _D)),
        b2g=jnp.broadcast_to(gp["l2"]["b"].reshape(1, 1), (1, _D)),
    )
    node = dict(
        wh=np_["l1"]["W"][:_D], wa=np_["l1"]["W"][_D:],
        b1=np_["l1"]["b"].reshape(1, _D),
        g1=np_["ln_g"].reshape(1, _D), t1=np_["ln_b"].reshape(1, _D),
        w2=np_["l2"]["W"], b2=np_["l2"]["b"].reshape(1, _D),
    )
    edge = dict(
        we=ep["l1"]["W"][:_D], ws=ep["l1"]["W"][_D:2 * _D],
        wd=ep["l1"]["W"][2 * _D:],
        b1=ep["l1"]["b"].reshape(1, _D),
        g1=ep["ln_g"].reshape(1, _D), t1=ep["ln_b"].reshape(1, _D),
        w2=ep["l2"]["W"], b2=ep["l2"]["b"].reshape(1, _D),
    )
    return msg, node, edge


def kernel(x, edge_index, edge_attr, params):
    src = edge_index[0]
    dst = edge_index[1]
    dst_a, dst_b = dst[:_EH], dst[_EH:]
    idx_a = jnp.concatenate([src[:_EH], dst_a])
    idx_b = jnp.concatenate([src[_EH:], dst_b])
    zer = jnp.zeros((_N, _D), jnp.float32)

    h = _linear_pl(x, params["node_enc"]["W"], params["node_enc"]["b"], _BN)
    e_a = _linear_pl(edge_attr[:_EH], params["edge_enc"]["W"],
                     params["edge_enc"]["b"], _BE)
    e_b = _linear_pl(edge_attr[_EH:], params["edge_enc"]["W"],
                     params["edge_enc"]["b"], _BE)

    g_a = _sc_gather(h, idx_a)
    g_b = _sc_gather(h, idx_b)
    for lp in params["layers"]:
        msg, node, edge = _pack_layer(lp)
        m_a = _message_pl(g_a, e_a, **msg)
        parts_a = _sc_scatter(m_a, dst_a, zer)
        m_b = _message_pl(g_b, e_b, **msg)
        parts_b = _sc_scatter(m_b, dst_b, zer)
        h = _node_pl(h, parts_a, parts_b, **node)
        g_a = _sc_gather(h, idx_a)
        e_a = _edgeupd_pl(e_a, g_a, **edge)
        g_b = _sc_gather(h, idx_b)
        e_b = _edgeupd_pl(e_b, g_b, **edge)

    x_out = _linear_pl(h, params["node_dec"]["W"], params["node_dec"]["b"], _BN)
    e_out_a = _linear_pl(e_a, params["edge_dec"]["W"],
                         params["edge_dec"]["b"], _BE)
    e_out_b = _linear_pl(e_b, params["edge_dec"]["W"],
                         params["edge_dec"]["b"], _BE)
    return (x_out, jnp.concatenate([e_out_a, e_out_b], axis=0))


# speedup vs baseline: 1.3877x; 1.0035x over previous
import functools

import jax
import jax.numpy as jnp
from jax import lax
from jax.experimental import pallas as pl
from jax.experimental.pallas import tpu as pltpu
from jax.experimental.pallas import tpu_sc as plsc

_N = 10000
_E = 320000
_EH = _E // 2
_D = 128
_LN_EPS = 1e-5

_BE = 8000
_BN = 2000
_NW = 32
_GCH = 80
_SCH = 40


def _sig(x):
    return 0.5 * jnp.tanh(0.5 * x) + 0.5


def _ln_silu(h, g, b):
    mu = jnp.mean(h, axis=-1, keepdims=True)
    c = h - mu
    var = jnp.mean(c * c, axis=-1, keepdims=True)
    hn = c * lax.rsqrt(var + _LN_EPS) * g + b
    return hn * _sig(hn)


def _const_spec(shape):
    return pl.BlockSpec(shape, lambda i: tuple(0 for _ in shape))


def _linear_pl(x, w, b, block):
    n, k = x.shape
    m = w.shape[1]

    def body(x_ref, w_ref, b_ref, o_ref):
        o_ref[...] = (
            jnp.dot(x_ref[...], w_ref[...], preferred_element_type=jnp.float32)
            + b_ref[...])

    return pl.pallas_call(
        body,
        grid=(n // block,),
        in_specs=[pl.BlockSpec((block, k), lambda i: (i, 0)),
                  _const_spec((k, m)),
                  _const_spec((1, m))],
        out_specs=pl.BlockSpec((block, m), lambda i: (i, 0)),
        out_shape=jax.ShapeDtypeStruct((n, m), jnp.float32),
    )(x, w, b.reshape(1, m))


def _message_pl(g, e, wd, ws, we, b1, g1, t1, w2m, b2m, w2g, b2g, goff=0):
    nb = _EH // _BE

    def body(hd_ref, hs_ref, e_ref, wd_, ws_, we_, b1_, g1_, t1_,
             w2m_, b2m_, w2g_, b2g_, o_ref):
        h1 = (jnp.dot(hd_ref[...], wd_[...], preferred_element_type=jnp.float32)
              + jnp.dot(hs_ref[...], ws_[...], preferred_element_type=jnp.float32)
              + jnp.dot(e_ref[...], we_[...], preferred_element_type=jnp.float32)
              + b1_[...])
        hm = _ln_silu(h1[:, :_D], g1_[:, :_D], t1_[:, :_D])
        hg = _ln_silu(h1[:, _D:], g1_[:, _D:], t1_[:, _D:])
        msg = jnp.dot(hm, w2m_[...], preferred_element_type=jnp.float32) + b2m_[...]
        gl = jnp.dot(hg, w2g_[...], preferred_element_type=jnp.float32) + b2g_[...]
        o_ref[...] = _sig(gl) * msg

    return pl.pallas_call(
        body,
        grid=(nb,),
        in_specs=[pl.BlockSpec((_BE, _D), lambda i: (i + goff + nb, 0)),
                  pl.BlockSpec((_BE, _D), lambda i: (i + goff, 0)),
                  pl.BlockSpec((_BE, _D), lambda i: (i, 0)),
                  _const_spec((_D, 2 * _D)), _const_spec((_D, 2 * _D)),
                  _const_spec((_D, 2 * _D)), _const_spec((1, 2 * _D)),
                  _const_spec((1, 2 * _D)), _const_spec((1, 2 * _D)),
                  _const_spec((_D, _D)), _const_spec((1, _D)),
                  _const_spec((_D, _D)), _const_spec((1, _D))],
        out_specs=pl.BlockSpec((_BE, _D), lambda i: (i, 0)),
        out_shape=jax.ShapeDtypeStruct((_EH, _D), jnp.float32),
    )(g, g, e, wd, ws, we, b1, g1, t1, w2m, b2m, w2g, b2g)


def _node_pl(h, pa, pb, wh, wa, b1, g1, t1, w2, b2):
    nb = _N // _BN

    def body(h_ref, pa0_ref, pa1_ref, pb0_ref, pb1_ref,
             wh_, wa_, b1_, g1_, t1_, w2_, b2_, o_ref):
        aggr = (pa0_ref[...] + pa1_ref[...]) + (pb0_ref[...] + pb1_ref[...])
        h1 = (jnp.dot(h_ref[...], wh_[...], preferred_element_type=jnp.float32)
              + jnp.dot(aggr, wa_[...], preferred_element_type=jnp.float32)
              + b1_[...])
        h1 = _ln_silu(h1, g1_[...], t1_[...])
        o_ref[...] = (jnp.dot(h1, w2_[...], preferred_element_type=jnp.float32)
                      + b2_[...] + h_ref[...])

    return pl.pallas_call(
        body,
        grid=(nb,),
        in_specs=[pl.BlockSpec((_BN, _D), lambda i: (i, 0)),
                  pl.BlockSpec((_BN, _D), lambda i: (i, 0)),
                  pl.BlockSpec((_BN, _D), lambda i: (i + nb, 0)),
                  pl.BlockSpec((_BN, _D), lambda i: (i, 0)),
                  pl.BlockSpec((_BN, _D), lambda i: (i + nb, 0)),
                  _const_spec((_D, _D)), _const_spec((_D, _D)),
                  _const_spec((1, _D)), _const_spec((1, _D)),
                  _const_spec((1, _D)), _const_spec((_D, _D)),
                  _const_spec((1, _D))],
        out_specs=pl.BlockSpec((_BN, _D), lambda i: (i, 0)),
        out_shape=jax.ShapeDtypeStruct((_N, _D), jnp.float32),
    )(h, pa, pa, pb, pb, wh, wa, b1, g1, t1, w2, b2)


def _edgeupd_pl(e, g, we, ws, wd, b1, g1, t1, w2, b2, dec=None):
    nb = _EH // _BE
    md = dec[0].shape[1] if dec is not None else 0

    def body(e_ref, ns_ref, nd_ref, we_, ws_, wd_, b1_, g1_, t1_, w2_, b2_,
             *rest):
        h1 = (jnp.dot(e_ref[...], we_[...], preferred_element_type=jnp.float32)
              + jnp.dot(ns_ref[...], ws_[...], preferred_element_type=jnp.float32)
              + jnp.dot(nd_ref[...], wd_[...], preferred_element_type=jnp.float32)
              + b1_[...])
        h1 = _ln_silu(h1, g1_[...], t1_[...])
        o = (jnp.dot(h1, w2_[...], preferred_element_type=jnp.float32)
             + b2_[...] + e_ref[...])
        if dec is None:
            rest[-1][...] = o
        else:
            wdec_, bdec_, o_ref, od_ref = rest
            o_ref[...] = o
            od_ref[...] = (jnp.dot(o, wdec_[...],
                                   preferred_element_type=jnp.float32)
                           + bdec_[...])

    in_specs = [pl.BlockSpec((_BE, _D), lambda i: (i, 0)),
                pl.BlockSpec((_BE, _D), lambda i: (i, 0)),
                pl.BlockSpec((_BE, _D), lambda i: (i + nb, 0)),
                _const_spec((_D, _D)), _const_spec((_D, _D)),
                _const_spec((_D, _D)), _const_spec((1, _D)),
                _const_spec((1, _D)), _const_spec((1, _D)),
                _const_spec((_D, _D)), _const_spec((1, _D))]
    args = [e, g, g, we, ws, wd, b1, g1, t1, w2, b2]
    out_specs = [pl.BlockSpec((_BE, _D), lambda i: (i, 0))]
    out_shape = [jax.ShapeDtypeStruct((_EH, _D), jnp.float32)]
    if dec is not None:
        in_specs += [_const_spec((_D, md)), _const_spec((1, md))]
        args += [dec[0], dec[1].reshape(1, md)]
        out_specs.append(pl.BlockSpec((_BE, md), lambda i: (i, 0)))
        out_shape.append(jax.ShapeDtypeStruct((_EH, md), jnp.float32))
    res = pl.pallas_call(
        body,
        grid=(nb,),
        in_specs=in_specs,
        out_specs=out_specs,
        out_shape=out_shape,
    )(*args)
    return res if dec is not None else res[0]


def _sc_gather(h, idx):
    b = idx.shape[0]
    per_w = b // _NW
    nch = per_w // _GCH
    nbuf = 4
    lead = nch % nbuf
    mesh = plsc.VectorSubcoreMesh(core_axis_name="c", subcore_axis_name="s")

    @functools.partial(
        pl.kernel, mesh=mesh,
        out_type=jax.ShapeDtypeStruct((b, _D), jnp.float32),
        scratch_types=[pltpu.VMEM((per_w,), jnp.int32)]
        + [pltpu.VMEM((_GCH, _D), jnp.float32)] * nbuf
        + [pltpu.SemaphoreType.DMA] * (2 * nbuf),
    )
    def k(h_hbm, idx_hbm, out_hbm, idx_v, *bufs):
        rows = bufs[:nbuf]
        gsem = bufs[nbuf:2 * nbuf]
        wsem = bufs[2 * nbuf:]
        wid = lax.axis_index("s") * 2 + lax.axis_index("c")
        base = wid * per_w
        pltpu.sync_copy(idx_hbm.at[pl.ds(base, per_w)], idx_v)

        for t in range(lead):
            pltpu.async_copy(h_hbm.at[idx_v.at[pl.ds(t * _GCH, _GCH)]],
                             rows[t], gsem[t]).wait()
            pltpu.sync_copy(rows[t],
                            out_hbm.at[pl.ds(base + t * _GCH, _GCH)])

        def outer(j, carry):
            for t in range(nbuf):
                i = lead + j * nbuf + t

                @pl.when(j > 0)
                def _():
                    pltpu.make_async_copy(
                        rows[t], out_hbm.at[pl.ds(base + i * _GCH, _GCH)],
                        wsem[t]).wait()

                pltpu.async_copy(
                    h_hbm.at[idx_v.at[pl.ds(i * _GCH, _GCH)]], rows[t],
                    gsem[t])
            for t in range(nbuf):
                i = lead + j * nbuf + t
                pltpu.make_async_copy(
                    h_hbm.at[idx_v.at[pl.ds(i * _GCH, _GCH)]], rows[t],
                    gsem[t]).wait()
                pltpu.async_copy(rows[t],
                                 out_hbm.at[pl.ds(base + i * _GCH, _GCH)],
                                 wsem[t])
            return carry

        lax.fori_loop(0, (nch - lead) // nbuf, outer, 0)
        for t in range(nbuf):
            pltpu.make_async_copy(
                rows[t], out_hbm.at[pl.ds(base, _GCH)], wsem[t]).wait()

    return k(h, idx)


def _sc_scatter(m, dsth, zer):
    per_w = _EH // _NW
    nch = per_w // _SCH
    stripe = 624
    tail = _N - 16 * stripe
    mesh = plsc.VectorSubcoreMesh(core_axis_name="c", subcore_axis_name="s")

    nbuf = 4

    @functools.partial(
        pl.kernel, mesh=mesh,
        out_type=jax.ShapeDtypeStruct((2 * _N, _D), jnp.float32),
        scratch_types=[pltpu.VMEM_SHARED((_N, _D), jnp.float32)]
        + [pltpu.VMEM((_SCH,), jnp.int32)] * nbuf
        + [pltpu.VMEM((_SCH, _D), jnp.float32)] * nbuf
        + [pltpu.SemaphoreType.DMA] * (2 * nbuf),
    )
    def k(m_hbm, dst_hbm, zer_hbm, out_hbm, acc, *bufs):
        idx_b = bufs[:nbuf]
        rows_b = bufs[nbuf:2 * nbuf]
        isem = bufs[2 * nbuf:3 * nbuf]
        msem = bufs[3 * nbuf:]
        cid = lax.axis_index("c")
        sid = lax.axis_index("s")
        wid = sid * 2 + cid
        pltpu.sync_copy(zer_hbm.at[pl.ds(sid * stripe, stripe)],
                        acc.at[pl.ds(sid * stripe, stripe)])

        @pl.when(sid == 15)
        def _():
            pltpu.sync_copy(zer_hbm.at[pl.ds(16 * stripe, tail)],
                            acc.at[pl.ds(16 * stripe, tail)])

        plsc.subcore_barrier()

        base = wid * per_w

        lead = nch % nbuf
        for t in range(lead):
            off = base + t * _SCH
            pltpu.sync_copy(dst_hbm.at[pl.ds(off, _SCH)], idx_b[t])
            pltpu.sync_copy(m_hbm.at[pl.ds(off, _SCH)], rows_b[t])
            pltpu.sync_copy(rows_b[t], acc.at[idx_b[t]], add=True)

        def outer(j, carry):
            for t in range(nbuf):
                i = lead + j * nbuf + t
                off = base + i * _SCH
                pltpu.async_copy(dst_hbm.at[pl.ds(off, _SCH)], idx_b[t],
                                 isem[t])
                pltpu.async_copy(m_hbm.at[pl.ds(off, _SCH)], rows_b[t],
                                 msem[t])
            for t in range(nbuf):
                i = lead + j * nbuf + t
                off = base + i * _SCH
                pltpu.make_async_copy(dst_hbm.at[pl.ds(off, _SCH)], idx_b[t],
                                      isem[t]).wait()
                pltpu.make_async_copy(m_hbm.at[pl.ds(off, _SCH)], rows_b[t],
                                      msem[t]).wait()
                pltpu.sync_copy(rows_b[t], acc.at[idx_b[t]], add=True)
            return carry

        lax.fori_loop(0, (nch - lead) // nbuf, outer, 0)
        plsc.subcore_barrier()
        pltpu.sync_copy(
            acc.at[pl.ds(sid * stripe, stripe)],
            out_hbm.at[pl.ds(cid * _N + sid * stripe, stripe)])

        @pl.when(sid == 15)
        def _():
            pltpu.sync_copy(
                acc.at[pl.ds(16 * stripe, tail)],
                out_hbm.at[pl.ds(cid * _N + 16 * stripe, tail)])

    return k(m, dsth, zer)


def _pack_layer(lp):
    mp, gp, np_, ep = lp["message"], lp["gate"], lp["node"], lp["edge"]
    w1 = jnp.concatenate([mp["l1"]["W"], gp["l1"]["W"]], axis=1)
    msg = dict(
        wd=w1[:_D], ws=w1[_D:2 * _D], we=w1[2 * _D:],
        b1=jnp.concatenate([mp["l1"]["b"], gp["l1"]["b"]]).reshape(1, 2 * _D),
        g1=jnp.concatenate([mp["ln_g"], gp["ln_g"]]).reshape(1, 2 * _D),
        t1=jnp.concatenate([mp["ln_b"], gp["ln_b"]]).reshape(1, 2 * _D),
        w2m=mp["l2"]["W"], b2m=mp["l2"]["b"].reshape(1, _D),
        w2g=jnp.broadcast_to(gp["l2"]["W"], (_D, _D)),
        b2g=jnp.broadcast_to(gp["l2"]["b"].reshape(1, 1), (1, _D)),
    )
    node = dict(
        wh=np_["l1"]["W"][:_D], wa=np_["l1"]["W"][_D:],
        b1=np_["l1"]["b"].reshape(1, _D),
        g1=np_["ln_g"].reshape(1, _D), t1=np_["ln_b"].reshape(1, _D),
        w2=np_["l2"]["W"], b2=np_["l2"]["b"].reshape(1, _D),
    )
    edge = dict(
        we=ep["l1"]["W"][:_D], ws=ep["l1"]["W"][_D:2 * _D],
        wd=ep["l1"]["W"][2 * _D:],
        b1=ep["l1"]["b"].reshape(1, _D),
        g1=ep["ln_g"].reshape(1, _D), t1=ep["ln_b"].reshape(1, _D),
        w2=ep["l2"]["W"], b2=ep["l2"]["b"].reshape(1, _D),
    )
    return msg, node, edge


def kernel(x, edge_index, edge_attr, params):
    src = edge_index[0]
    dst = edge_index[1]
    dst_a, dst_b = dst[:_EH], dst[_EH:]
    idx_a = jnp.concatenate([src[:_EH], dst_a])
    idx_b = jnp.concatenate([src[_EH:], dst_b])
    idx_all = jnp.concatenate([idx_a, idx_b])
    zer = jnp.zeros((_N, _D), jnp.float32)

    h = _linear_pl(x, params["node_enc"]["W"], params["node_enc"]["b"], _BN)
    e_a = _linear_pl(edge_attr[:_EH], params["edge_enc"]["W"],
                     params["edge_enc"]["b"], _BE)
    e_b = _linear_pl(edge_attr[_EH:], params["edge_enc"]["W"],
                     params["edge_enc"]["b"], _BE)

    g_a = g_b = _sc_gather(h, idx_all)
    goff_b = 2 * (_EH // _BE)
    nlay = len(params["layers"])
    for li, lp in enumerate(params["layers"]):
        msg, node, edge = _pack_layer(lp)
        m_a = _message_pl(g_a, e_a, **msg, goff=0)
        parts_a = _sc_scatter(m_a, dst_a, zer)
        m_b = _message_pl(g_b, e_b, **msg, goff=goff_b if li == 0 else 0)
        parts_b = _sc_scatter(m_b, dst_b, zer)
        h = _node_pl(h, parts_a, parts_b, **node)
        dec = ((params["edge_dec"]["W"], params["edge_dec"]["b"])
               if li == nlay - 1 else None)
        g_a = _sc_gather(h, idx_a)
        ua = _edgeupd_pl(e_a, g_a, **edge, dec=dec)
        g_b = _sc_gather(h, idx_b)
        ub = _edgeupd_pl(e_b, g_b, **edge, dec=dec)
        if dec is None:
            e_a, e_b = ua, ub
        else:
            (e_a, e_out_a), (e_b, e_out_b) = ua, ub

    x_out = _linear_pl(h, params["node_dec"]["W"], params["node_dec"]["b"], _BN)
    return (x_out, jnp.concatenate([e_out_a, e_out_b], axis=0))


# speedup vs baseline: 1.4279x; 1.0290x over previous
import functools

import jax
import jax.numpy as jnp
from jax import lax
from jax.experimental import pallas as pl
from jax.experimental.pallas import tpu as pltpu
from jax.experimental.pallas import tpu_sc as plsc

_N = 10000
_E = 320000
_EH = _E // 2
_D = 128
_LN_EPS = 1e-5

_BE = 8000
_BN = 2000
_NW = 32
_GCH = 80
_SCH = 40


def _sig(x):
    return 0.5 * jnp.tanh(0.5 * x) + 0.5


def _ln_silu(h, g, b):
    mu = jnp.mean(h, axis=-1, keepdims=True)
    c = h - mu
    var = jnp.mean(c * c, axis=-1, keepdims=True)
    hn = c * lax.rsqrt(var + _LN_EPS) * g + b
    return hn * _sig(hn)


def _const_spec(shape):
    return pl.BlockSpec(shape, lambda i: tuple(0 for _ in shape))


def _linear_pl(x, w, b, block):
    n, k = x.shape
    m = w.shape[1]

    def body(x_ref, w_ref, b_ref, o_ref):
        o_ref[...] = (
            jnp.dot(x_ref[...], w_ref[...], preferred_element_type=jnp.float32)
            + b_ref[...])

    return pl.pallas_call(
        body,
        grid=(n // block,),
        in_specs=[pl.BlockSpec((block, k), lambda i: (i, 0)),
                  _const_spec((k, m)),
                  _const_spec((1, m))],
        out_specs=pl.BlockSpec((block, m), lambda i: (i, 0)),
        out_shape=jax.ShapeDtypeStruct((n, m), jnp.float32),
    )(x, w, b.reshape(1, m))


def _message_pl(g, e, wd, ws, we, b1, g1, t1, w2m, b2m, w2g, b2g, goff=0):
    nb = _EH // _BE
    ke = e.shape[1]

    def body(hd_ref, hs_ref, e_ref, wd_, ws_, we_, b1_, g1_, t1_,
             w2m_, b2m_, w2g_, b2g_, o_ref):
        h1 = (jnp.dot(hd_ref[...], wd_[...], preferred_element_type=jnp.float32)
              + jnp.dot(hs_ref[...], ws_[...], preferred_element_type=jnp.float32)
              + jnp.dot(e_ref[...], we_[...], preferred_element_type=jnp.float32)
              + b1_[...])
        hm = _ln_silu(h1[:, :_D], g1_[:, :_D], t1_[:, :_D])
        hg = _ln_silu(h1[:, _D:], g1_[:, _D:], t1_[:, _D:])
        msg = jnp.dot(hm, w2m_[...], preferred_element_type=jnp.float32) + b2m_[...]
        gl = jnp.dot(hg, w2g_[...], preferred_element_type=jnp.float32) + b2g_[...]
        o_ref[...] = _sig(gl) * msg

    return pl.pallas_call(
        body,
        grid=(nb,),
        in_specs=[pl.BlockSpec((_BE, _D), lambda i: (i + goff + nb, 0)),
                  pl.BlockSpec((_BE, _D), lambda i: (i + goff, 0)),
                  pl.BlockSpec((_BE, ke), lambda i: (i, 0)),
                  _const_spec((_D, 2 * _D)), _const_spec((_D, 2 * _D)),
                  _const_spec((ke, 2 * _D)), _const_spec((1, 2 * _D)),
                  _const_spec((1, 2 * _D)), _const_spec((1, 2 * _D)),
                  _const_spec((_D, _D)), _const_spec((1, _D)),
                  _const_spec((_D, _D)), _const_spec((1, _D))],
        out_specs=pl.BlockSpec((_BE, _D), lambda i: (i, 0)),
        out_shape=jax.ShapeDtypeStruct((_EH, _D), jnp.float32),
    )(g, g, e, wd, ws, we, b1, g1, t1, w2m, b2m, w2g, b2g)


def _node_pl(h, pa, pb, wh, wa, b1, g1, t1, w2, b2):
    nb = _N // _BN

    def body(h_ref, pa0_ref, pa1_ref, pb0_ref, pb1_ref,
             wh_, wa_, b1_, g1_, t1_, w2_, b2_, o_ref):
        aggr = (pa0_ref[...] + pa1_ref[...]) + (pb0_ref[...] + pb1_ref[...])
        h1 = (jnp.dot(h_ref[...], wh_[...], preferred_element_type=jnp.float32)
              + jnp.dot(aggr, wa_[...], preferred_element_type=jnp.float32)
              + b1_[...])
        h1 = _ln_silu(h1, g1_[...], t1_[...])
        o_ref[...] = (jnp.dot(h1, w2_[...], preferred_element_type=jnp.float32)
                      + b2_[...] + h_ref[...])

    return pl.pallas_call(
        body,
        grid=(nb,),
        in_specs=[pl.BlockSpec((_BN, _D), lambda i: (i, 0)),
                  pl.BlockSpec((_BN, _D), lambda i: (i, 0)),
                  pl.BlockSpec((_BN, _D), lambda i: (i + nb, 0)),
                  pl.BlockSpec((_BN, _D), lambda i: (i, 0)),
                  pl.BlockSpec((_BN, _D), lambda i: (i + nb, 0)),
                  _const_spec((_D, _D)), _const_spec((_D, _D)),
                  _const_spec((1, _D)), _const_spec((1, _D)),
                  _const_spec((1, _D)), _const_spec((_D, _D)),
                  _const_spec((1, _D))],
        out_specs=pl.BlockSpec((_BN, _D), lambda i: (i, 0)),
        out_shape=jax.ShapeDtypeStruct((_N, _D), jnp.float32),
    )(h, pa, pa, pb, pb, wh, wa, b1, g1, t1, w2, b2)


def _edgeupd_pl(e, g, we, ws, wd, b1, g1, t1, w2, b2, dec=None, enc=None):
    nb = _EH // _BE
    ke = e.shape[1]
    md = dec[0].shape[1] if dec is not None else 0

    def body(e_ref, ns_ref, nd_ref, we_, ws_, wd_, b1_, g1_, t1_, w2_, b2_,
             *rest):
        h1 = (jnp.dot(e_ref[...], we_[...], preferred_element_type=jnp.float32)
              + jnp.dot(ns_ref[...], ws_[...], preferred_element_type=jnp.float32)
              + jnp.dot(nd_ref[...], wd_[...], preferred_element_type=jnp.float32)
              + b1_[...])
        h1 = _ln_silu(h1, g1_[...], t1_[...])
        o2 = (jnp.dot(h1, w2_[...], preferred_element_type=jnp.float32)
              + b2_[...])
        if enc is not None:
            wenc_, benc_, o_ref = rest
            o_ref[...] = o2 + (jnp.dot(e_ref[...], wenc_[...],
                                       preferred_element_type=jnp.float32)
                               + benc_[...])
        elif dec is not None:
            wdec_, bdec_, o_ref, od_ref = rest
            o = o2 + e_ref[...]
            o_ref[...] = o
            od_ref[...] = (jnp.dot(o, wdec_[...],
                                   preferred_element_type=jnp.float32)
                           + bdec_[...])
        else:
            rest[-1][...] = o2 + e_ref[...]

    in_specs = [pl.BlockSpec((_BE, ke), lambda i: (i, 0)),
                pl.BlockSpec((_BE, _D), lambda i: (i, 0)),
                pl.BlockSpec((_BE, _D), lambda i: (i + nb, 0)),
                _const_spec((ke, _D)), _const_spec((_D, _D)),
                _const_spec((_D, _D)), _const_spec((1, _D)),
                _const_spec((1, _D)), _const_spec((1, _D)),
                _const_spec((_D, _D)), _const_spec((1, _D))]
    args = [e, g, g, we, ws, wd, b1, g1, t1, w2, b2]
    out_specs = [pl.BlockSpec((_BE, _D), lambda i: (i, 0))]
    out_shape = [jax.ShapeDtypeStruct((_EH, _D), jnp.float32)]
    if enc is not None:
        in_specs += [_const_spec((ke, _D)), _const_spec((1, _D))]
        args += [enc[0], enc[1].reshape(1, _D)]
    if dec is not None:
        in_specs += [_const_spec((_D, md)), _const_spec((1, md))]
        args += [dec[0], dec[1].reshape(1, md)]
        out_specs.append(pl.BlockSpec((_BE, md), lambda i: (i, 0)))
        out_shape.append(jax.ShapeDtypeStruct((_EH, md), jnp.float32))
    res = pl.pallas_call(
        body,
        grid=(nb,),
        in_specs=in_specs,
        out_specs=out_specs,
        out_shape=out_shape,
    )(*args)
    return res if dec is not None else res[0]


def _sc_gather(h, idx):
    b = idx.shape[0]
    per_w = b // _NW
    nch = per_w // _GCH
    nbuf = 4
    lead = nch % nbuf
    mesh = plsc.VectorSubcoreMesh(core_axis_name="c", subcore_axis_name="s")

    @functools.partial(
        pl.kernel, mesh=mesh,
        out_type=jax.ShapeDtypeStruct((b, _D), jnp.float32),
        scratch_types=[pltpu.VMEM((per_w,), jnp.int32)]
        + [pltpu.VMEM((_GCH, _D), jnp.float32)] * nbuf
        + [pltpu.SemaphoreType.DMA] * (2 * nbuf),
    )
    def k(h_hbm, idx_hbm, out_hbm, idx_v, *bufs):
        rows = bufs[:nbuf]
        gsem = bufs[nbuf:2 * nbuf]
        wsem = bufs[2 * nbuf:]
        wid = lax.axis_index("s") * 2 + lax.axis_index("c")
        base = wid * per_w
        pltpu.sync_copy(idx_hbm.at[pl.ds(base, per_w)], idx_v)

        for t in range(lead):
            pltpu.async_copy(h_hbm.at[idx_v.at[pl.ds(t * _GCH, _GCH)]],
                             rows[t], gsem[t]).wait()
            pltpu.sync_copy(rows[t],
                            out_hbm.at[pl.ds(base + t * _GCH, _GCH)])

        def outer(j, carry):
            for t in range(nbuf):
                i = lead + j * nbuf + t

                @pl.when(j > 0)
                def _():
                    pltpu.make_async_copy(
                        rows[t], out_hbm.at[pl.ds(base + i * _GCH, _GCH)],
                        wsem[t]).wait()

                pltpu.async_copy(
                    h_hbm.at[idx_v.at[pl.ds(i * _GCH, _GCH)]], rows[t],
                    gsem[t])
            for t in range(nbuf):
                i = lead + j * nbuf + t
                pltpu.make_async_copy(
                    h_hbm.at[idx_v.at[pl.ds(i * _GCH, _GCH)]], rows[t],
                    gsem[t]).wait()
                pltpu.async_copy(rows[t],
                                 out_hbm.at[pl.ds(base + i * _GCH, _GCH)],
                                 wsem[t])
            return carry

        lax.fori_loop(0, (nch - lead) // nbuf, outer, 0)
        for t in range(nbuf):
            pltpu.make_async_copy(
                rows[t], out_hbm.at[pl.ds(base, _GCH)], wsem[t]).wait()

    return k(h, idx)


def _sc_scatter(m, dsth, zer):
    per_w = _EH // _NW
    nch = per_w // _SCH
    stripe = 624
    tail = _N - 16 * stripe
    mesh = plsc.VectorSubcoreMesh(core_axis_name="c", subcore_axis_name="s")

    nbuf = 4

    @functools.partial(
        pl.kernel, mesh=mesh,
        out_type=jax.ShapeDtypeStruct((2 * _N, _D), jnp.float32),
        scratch_types=[pltpu.VMEM_SHARED((_N, _D), jnp.float32)]
        + [pltpu.VMEM((_SCH,), jnp.int32)] * nbuf
        + [pltpu.VMEM((_SCH, _D), jnp.float32)] * nbuf
        + [pltpu.SemaphoreType.DMA] * (2 * nbuf),
    )
    def k(m_hbm, dst_hbm, zer_hbm, out_hbm, acc, *bufs):
        idx_b = bufs[:nbuf]
        rows_b = bufs[nbuf:2 * nbuf]
        isem = bufs[2 * nbuf:3 * nbuf]
        msem = bufs[3 * nbuf:]
        cid = lax.axis_index("c")
        sid = lax.axis_index("s")
        wid = sid * 2 + cid
        pltpu.sync_copy(zer_hbm.at[pl.ds(sid * stripe, stripe)],
                        acc.at[pl.ds(sid * stripe, stripe)])

        @pl.when(sid == 15)
        def _():
            pltpu.sync_copy(zer_hbm.at[pl.ds(16 * stripe, tail)],
                            acc.at[pl.ds(16 * stripe, tail)])

        plsc.subcore_barrier()

        base = wid * per_w

        lead = nch % nbuf
        for t in range(lead):
            off = base + t * _SCH
            pltpu.sync_copy(dst_hbm.at[pl.ds(off, _SCH)], idx_b[t])
            pltpu.sync_copy(m_hbm.at[pl.ds(off, _SCH)], rows_b[t])
            pltpu.sync_copy(rows_b[t], acc.at[idx_b[t]], add=True)

        def outer(j, carry):
            for t in range(nbuf):
                i = lead + j * nbuf + t
                off = base + i * _SCH
                pltpu.async_copy(dst_hbm.at[pl.ds(off, _SCH)], idx_b[t],
                                 isem[t])
                pltpu.async_copy(m_hbm.at[pl.ds(off, _SCH)], rows_b[t],
                                 msem[t])
            for t in range(nbuf):
                i = lead + j * nbuf + t
                off = base + i * _SCH
                pltpu.make_async_copy(dst_hbm.at[pl.ds(off, _SCH)], idx_b[t],
                                      isem[t]).wait()
                pltpu.make_async_copy(m_hbm.at[pl.ds(off, _SCH)], rows_b[t],
                                      msem[t]).wait()
                pltpu.sync_copy(rows_b[t], acc.at[idx_b[t]], add=True)
            return carry

        lax.fori_loop(0, (nch - lead) // nbuf, outer, 0)
        plsc.subcore_barrier()
        pltpu.sync_copy(
            acc.at[pl.ds(sid * stripe, stripe)],
            out_hbm.at[pl.ds(cid * _N + sid * stripe, stripe)])

        @pl.when(sid == 15)
        def _():
            pltpu.sync_copy(
                acc.at[pl.ds(16 * stripe, tail)],
                out_hbm.at[pl.ds(cid * _N + 16 * stripe, tail)])

    return k(m, dsth, zer)


def _pack_layer(lp, enc_p=None):
    mp, gp, np_, ep = lp["message"], lp["gate"], lp["node"], lp["edge"]
    w1 = jnp.concatenate([mp["l1"]["W"], gp["l1"]["W"]], axis=1)
    msg_we = w1[2 * _D:]
    msg_b1 = jnp.concatenate([mp["l1"]["b"], gp["l1"]["b"]]).reshape(1, 2 * _D)
    edge_we = ep["l1"]["W"][:_D]
    edge_b1 = ep["l1"]["b"].reshape(1, _D)
    if enc_p is not None:
        wenc, benc = enc_p
        msg_b1 = msg_b1 + benc.reshape(1, _D) @ msg_we
        msg_we = wenc @ msg_we
        edge_b1 = edge_b1 + benc.reshape(1, _D) @ edge_we
        edge_we = wenc @ edge_we
    msg = dict(
        wd=w1[:_D], ws=w1[_D:2 * _D], we=msg_we,
        b1=msg_b1,
        g1=jnp.concatenate([mp["ln_g"], gp["ln_g"]]).reshape(1, 2 * _D),
        t1=jnp.concatenate([mp["ln_b"], gp["ln_b"]]).reshape(1, 2 * _D),
        w2m=mp["l2"]["W"], b2m=mp["l2"]["b"].reshape(1, _D),
        w2g=jnp.broadcast_to(gp["l2"]["W"], (_D, _D)),
        b2g=jnp.broadcast_to(gp["l2"]["b"].reshape(1, 1), (1, _D)),
    )
    node = dict(
        wh=np_["l1"]["W"][:_D], wa=np_["l1"]["W"][_D:],
        b1=np_["l1"]["b"].reshape(1, _D),
        g1=np_["ln_g"].reshape(1, _D), t1=np_["ln_b"].reshape(1, _D),
        w2=np_["l2"]["W"], b2=np_["l2"]["b"].reshape(1, _D),
    )
    edge = dict(
        we=edge_we, ws=ep["l1"]["W"][_D:2 * _D],
        wd=ep["l1"]["W"][2 * _D:],
        b1=edge_b1,
        g1=ep["ln_g"].reshape(1, _D), t1=ep["ln_b"].reshape(1, _D),
        w2=ep["l2"]["W"], b2=ep["l2"]["b"].reshape(1, _D),
        enc=enc_p,
    )
    return msg, node, edge


def kernel(x, edge_index, edge_attr, params):
    src = edge_index[0]
    dst = edge_index[1]
    dst_a, dst_b = dst[:_EH], dst[_EH:]
    idx_a = jnp.concatenate([src[:_EH], dst_a])
    idx_b = jnp.concatenate([src[_EH:], dst_b])
    idx_all = jnp.concatenate([idx_a, idx_b])
    zer = jnp.zeros((_N, _D), jnp.float32)

    h = _linear_pl(x, params["node_enc"]["W"], params["node_enc"]["b"], _BN)
    e_a = edge_attr[:_EH]
    e_b = edge_attr[_EH:]

    g_a = g_b = _sc_gather(h, idx_all)
    goff_b = 2 * (_EH // _BE)
    nlay = len(params["layers"])
    for li, lp in enumerate(params["layers"]):
        enc_p = ((params["edge_enc"]["W"], params["edge_enc"]["b"])
                 if li == 0 else None)
        msg, node, edge = _pack_layer(lp, enc_p)
        m_a = _message_pl(g_a, e_a, **msg, goff=0)
        parts_a = _sc_scatter(m_a, dst_a, zer)
        m_b = _message_pl(g_b, e_b, **msg, goff=goff_b if li == 0 else 0)
        parts_b = _sc_scatter(m_b, dst_b, zer)
        h = _node_pl(h, parts_a, parts_b, **node)
        dec = ((params["edge_dec"]["W"], params["edge_dec"]["b"])
               if li == nlay - 1 else None)
        g_a = _sc_gather(h, idx_a)
        ua = _edgeupd_pl(e_a, g_a, **edge, dec=dec)
        g_b = _sc_gather(h, idx_b)
        ub = _edgeupd_pl(e_b, g_b, **edge, dec=dec)
        if dec is None:
            e_a, e_b = ua, ub
        else:
            (e_a, e_out_a), (e_b, e_out_b) = ua, ub

    x_out = _linear_pl(h, params["node_dec"]["W"], params["node_dec"]["b"], _BN)
    return (x_out, jnp.concatenate([e_out_a, e_out_b], axis=0))
